# Initial kernel scaffold; baseline (speedup 1.0000x reference)
import jax
import jax.numpy as jnp
from jax.experimental import pallas as pl


def kernel(node_features, edge_features, senders, receivers, image_feature, params):
    raise NotImplementedError("write your pallas kernel here")


# speedup vs baseline: 3.9396x; 3.9396x over previous
import functools

import jax
import jax.numpy as jnp
from jax import lax
from jax.experimental import pallas as pl
from jax.experimental.pallas import tpu as pltpu
from jax.experimental.pallas import tpu_sc as plsc

N_NODES = 10000
N_SP = 10240
N_EDGES = 160000
NW = 32
CHUNK = 128
CH_PER_W = 40
E_PER_W = CHUNK * CH_PER_W
E_PAD = NW * E_PER_W
EB = 1280
NB = 2000
ZROWS = N_SP // 16
F32 = jnp.float32


def _ln_rows(x, g, beta):
    mu = jnp.mean(x, axis=-1, keepdims=True)
    var = jnp.mean((x - mu) ** 2, axis=-1, keepdims=True)
    return (x - mu) / jnp.sqrt(var + 1e-5) * g + beta


def _full(shape):
    nd = len(shape)
    return pl.BlockSpec(shape, lambda i: (0,) * nd)


def _enc_node(nf, img, p):
    (W1, b1), (W2, b2) = p["layers"]
    g, beta = p["ln"]
    W1a, W1b = W1[:128], W1[128:]

    def body(nf_ref, img_ref, w1a, w1b, b1_, w2, b2_, g_, be_, o_ref):
        h = nf_ref[...] @ w1a[...] + img_ref[...] @ w1b[...] + b1_[...]
        h = jnp.maximum(h, 0.0)
        y = h @ w2[...] + b2_[...]
        o_ref[...] = _ln_rows(y, g_[...], be_[...])

    return pl.pallas_call(
        body,
        grid=(N_NODES // NB,),
        in_specs=[
            pl.BlockSpec((NB, 128), lambda i: (i, 0)),
            _full((1, 512)), _full((128, 256)), _full((512, 256)),
            _full((256,)), _full((256, 128)), _full((128,)),
            _full((128,)), _full((128,)),
        ],
        out_specs=pl.BlockSpec((NB, 128), lambda i: (i, 0)),
        out_shape=jax.ShapeDtypeStruct((N_NODES, 128), F32),
    )(nf, img, W1a, W1b, b1, W2, b2, g, beta)


def _enc_edge(ef, p):
    (W1, b1), (W2, b2) = p["layers"]
    g, beta = p["ln"]

    def body(ef_ref, w1, b1_, w2, b2_, g_, be_, o_ref):
        h = ef_ref[...] @ w1[...] + b1_[...]
        h = jnp.maximum(h, 0.0)
        y = h @ w2[...] + b2_[...]
        o_ref[...] = _ln_rows(y, g_[...], be_[...])

    return pl.pallas_call(
        body,
        grid=(E_PAD // EB,),
        in_specs=[
            pl.BlockSpec((EB, 16), lambda i: (i, 0)),
            _full((16, 256)), _full((256,)), _full((256, 128)),
            _full((128,)), _full((128,)), _full((128,)),
        ],
        out_specs=pl.BlockSpec((EB, 128), lambda i: (i, 0)),
        out_shape=jax.ShapeDtypeStruct((E_PAD, 128), F32),
    )(ef, W1, b1, W2, b2, g, beta)


def _edge_tc(srows, rrows, elat, blk, want_resid):
    (W1, b1), (W2, b2), (W3, b3) = blk["edge"]["layers"]
    g, beta = blk["edge"]["ln"]
    aW, ab = blk["att"]
    W1s, W1r, W1e = W1[:128], W1[128:256], W1[256:]
    aWr = aW.reshape(1, 128)
    ab2 = ab.reshape(1, 1)

    def body(s_ref, r_ref, e_ref, w1s, w1r, w1e, b1_, w2, b2_, w3, b3_,
             g_, be_, aw, ab_, w_ref, x_ref, *res):
        h = (s_ref[...] @ w1s[...] + r_ref[...] @ w1r[...]
             + e_ref[...] @ w1e[...] + b1_[...])
        h = jnp.maximum(h, 0.0)
        h = jnp.maximum(h @ w2[...] + b2_[...], 0.0)
        ne = _ln_rows(h @ w3[...] + b3_[...], g_[...], be_[...])
        lg = jnp.sum(ne * aw[...], axis=-1, keepdims=True) + ab_[...]
        lg = jnp.where(lg >= 0, lg, 0.2 * lg)
        ex = jnp.exp(lg)
        w_ref[...] = ne * ex
        x_ref[...] = ex
        if want_resid:
            res[0][...] = ne + e_ref[...]

    out_shape = [jax.ShapeDtypeStruct((E_PAD, 128), F32),
                 jax.ShapeDtypeStruct((E_PAD, 1), F32)]
    out_specs = [pl.BlockSpec((EB, 128), lambda i: (i, 0)),
                 pl.BlockSpec((EB, 1), lambda i: (i, 0))]
    if want_resid:
        out_shape.append(jax.ShapeDtypeStruct((E_PAD, 128), F32))
        out_specs.append(pl.BlockSpec((EB, 128), lambda i: (i, 0)))

    return pl.pallas_call(
        body,
        grid=(E_PAD // EB,),
        in_specs=[
            pl.BlockSpec((EB, 128), lambda i: (i, 0)),
            pl.BlockSpec((EB, 128), lambda i: (i, 0)),
            pl.BlockSpec((EB, 128), lambda i: (i, 0)),
            _full((128, 128)), _full((128, 128)), _full((128, 128)),
            _full((128,)), _full((128, 128)), _full((128,)),
            _full((128, 128)), _full((128,)),
            _full((128,)), _full((128,)), _full((1, 128)), _full((1, 1)),
        ],
        out_specs=out_specs,
        out_shape=out_shape,
    )(srows, rrows, elat, W1s, W1r, W1e, b1, W2, b2, W3, b3, g, beta,
      aWr, ab2)


def _node_tc(nlat, a0, a1, s0, s1, p):
    (W1, b1), (W2, b2), (W3, b3) = p["layers"]
    g, beta = p["ln"]
    W1n, W1a = W1[:128], W1[128:]

    def body(n_ref, a0_, a1_, s0_, s1_, w1n, w1a, b1_, w2, b2_, w3, b3_,
             g_, be_, o_ref):
        n = n_ref[...]
        agg = (a0_[...] + a1_[...]) / (s0_[...] + s1_[...] + 1e-16)
        h = jnp.maximum(n @ w1n[...] + agg @ w1a[...] + b1_[...], 0.0)
        h = jnp.maximum(h @ w2[...] + b2_[...], 0.0)
        y = _ln_rows(h @ w3[...] + b3_[...], g_[...], be_[...])
        o_ref[...] = y + n

    return pl.pallas_call(
        body,
        grid=(N_NODES // NB,),
        in_specs=[
            pl.BlockSpec((NB, 128), lambda i: (i, 0)),
            pl.BlockSpec((NB, 128), lambda i: (i, 0)),
            pl.BlockSpec((NB, 128), lambda i: (i, 0)),
            pl.BlockSpec((NB, 1), lambda i: (i, 0)),
            pl.BlockSpec((NB, 1), lambda i: (i, 0)),
            _full((128, 128)), _full((128, 128)), _full((128,)),
            _full((128, 128)), _full((128,)), _full((128, 128)),
            _full((128,)), _full((128,)), _full((128,)),
        ],
        out_specs=pl.BlockSpec((NB, 128), lambda i: (i, 0)),
        out_shape=jax.ShapeDtypeStruct((N_NODES, 128), F32),
    )(nlat, a0, a1, s0, s1, W1n, W1a, b1, W2, b2, W3, b3, g, beta)


def _dec_tc(nlat, p):
    (W1, b1), (W2, b2), (W3, b3) = p["layers"]
    W3p = jnp.zeros((128, 128), F32).at[:, :W3.shape[1]].set(W3)
    b3p = jnp.zeros((128,), F32).at[:b3.shape[0]].set(b3)

    def body(n_ref, w1, b1_, w2, b2_, w3, b3_, o_ref):
        h = jnp.maximum(n_ref[...] @ w1[...] + b1_[...], 0.0)
        h = jnp.maximum(h @ w2[...] + b2_[...], 0.0)
        o_ref[...] = h @ w3[...] + b3_[...]

    return pl.pallas_call(
        body,
        grid=(N_NODES // NB,),
        in_specs=[
            pl.BlockSpec((NB, 128), lambda i: (i, 0)),
            _full((128, 128)), _full((128,)), _full((128, 128)),
            _full((128,)), _full((128, 128)), _full((128,)),
        ],
        out_specs=pl.BlockSpec((NB, 128), lambda i: (i, 0)),
        out_shape=jax.ShapeDtypeStruct((N_NODES, 128), F32),
    )(nlat, W1, b1, W2, b2, W3p, b3p)


def _sc_gather(table, ix_s3, ix_r3):
    mesh = plsc.VectorSubcoreMesh(core_axis_name="c", subcore_axis_name="s")

    @functools.partial(
        pl.kernel,
        out_type=(jax.ShapeDtypeStruct((E_PAD, 128), F32),
                  jax.ShapeDtypeStruct((E_PAD, 128), F32)),
        mesh=mesh,
        scratch_types=[
            pltpu.VMEM((CH_PER_W, CHUNK), jnp.int32),
            pltpu.VMEM((CH_PER_W, CHUNK), jnp.int32),
            pltpu.VMEM((CHUNK, 128), F32),
            pltpu.VMEM((CHUNK, 128), F32),
            pltpu.SemaphoreType.DMA,
            pltpu.SemaphoreType.DMA,
        ],
    )
    def k(table_hbm, ixs_hbm, ixr_hbm, os_hbm, or_hbm,
          ixs_v, ixr_v, bs, br, sem_s, sem_r):
        wid = lax.axis_index("s") * 2 + lax.axis_index("c")
        pltpu.sync_copy(ixs_hbm.at[wid], ixs_v)
        pltpu.sync_copy(ixr_hbm.at[wid], ixr_v)
        base = wid * E_PER_W

        def step(c, carry):
            cs = pltpu.async_copy(table_hbm.at[ixs_v.at[c]], bs, sem_s)
            cr = pltpu.async_copy(table_hbm.at[ixr_v.at[c]], br, sem_r)
            cs.wait()
            pltpu.sync_copy(bs, os_hbm.at[pl.ds(base + c * CHUNK, CHUNK)])
            cr.wait()
            pltpu.sync_copy(br, or_hbm.at[pl.ds(base + c * CHUNK, CHUNK)])
            return carry

        lax.fori_loop(0, CH_PER_W, step, 0)

    return k(table, ix_s3, ix_r3)


def _sc_scatter(wrows, expl3, ix_r3):
    mesh = plsc.VectorSubcoreMesh(core_axis_name="c", subcore_axis_name="s")

    @functools.partial(
        pl.kernel,
        out_type=(jax.ShapeDtypeStruct((N_SP, 128), F32),
                  jax.ShapeDtypeStruct((N_SP, 128), F32),
                  jax.ShapeDtypeStruct((N_SP,), F32),
                  jax.ShapeDtypeStruct((N_SP,), F32)),
        mesh=mesh,
        scratch_types=[
            pltpu.VMEM_SHARED((N_SP, 128), F32),
            pltpu.VMEM_SHARED((N_SP,), F32),
            pltpu.VMEM((CH_PER_W, CHUNK), jnp.int32),
            pltpu.VMEM((CH_PER_W, CHUNK), F32),
            pltpu.VMEM((CHUNK, 128), F32),
        ],
    )
    def k(w_hbm, ex_hbm, ix_hbm, a0_hbm, a1_hbm, s0_hbm, s1_hbm,
          spa, sps, ix_v, ex_v, buf):
        cid = lax.axis_index("c")
        sid = lax.axis_index("s")
        wid = sid * 2 + cid

        def zrow(i, carry):
            for j in range(8):
                buf[i, pl.ds(j * 16, 16)] = jnp.zeros((16,), F32)
            return carry

        lax.fori_loop(0, CHUNK, zrow, 0)
        z0 = sid * ZROWS
        for kk in range(ZROWS // CHUNK):
            pltpu.sync_copy(buf, spa.at[pl.ds(z0 + kk * CHUNK, CHUNK)])
            pltpu.sync_copy(buf.at[0], sps.at[pl.ds(z0 + kk * CHUNK, CHUNK)])
        plsc.subcore_barrier()

        pltpu.sync_copy(ix_hbm.at[wid], ix_v)
        pltpu.sync_copy(ex_hbm.at[wid], ex_v)
        base = wid * E_PER_W

        def step(c, carry):
            pltpu.sync_copy(w_hbm.at[pl.ds(base + c * CHUNK, CHUNK)], buf)
            pltpu.sync_copy(buf, spa.at[ix_v.at[c]], add=True)
            pltpu.sync_copy(ex_v.at[c], sps.at[ix_v.at[c]], add=True)
            return carry

        lax.fori_loop(0, CH_PER_W, step, 0)
        plsc.subcore_barrier()

        @pl.when(cid == 0)
        def _():
            pltpu.sync_copy(spa.at[pl.ds(z0, ZROWS)], a0_hbm.at[pl.ds(z0, ZROWS)])
            pltpu.sync_copy(sps.at[pl.ds(z0, ZROWS)], s0_hbm.at[pl.ds(z0, ZROWS)])

        @pl.when(cid == 1)
        def _():
            pltpu.sync_copy(spa.at[pl.ds(z0, ZROWS)], a1_hbm.at[pl.ds(z0, ZROWS)])
            pltpu.sync_copy(sps.at[pl.ds(z0, ZROWS)], s1_hbm.at[pl.ds(z0, ZROWS)])

    return k(wrows, expl3, ix_r3)


def kernel(node_features, edge_features, senders, receivers, image_feature,
           params):
    pad = E_PAD - N_EDGES
    s_pad = jnp.concatenate([senders, jnp.zeros((pad,), jnp.int32)])
    r_pad = jnp.concatenate([receivers, jnp.full((pad,), N_NODES, jnp.int32)])
    ix_s3 = s_pad.reshape(NW, CH_PER_W, CHUNK)
    ix_r3 = r_pad.reshape(NW, CH_PER_W, CHUNK)
    ef_pad = jnp.pad(edge_features, ((0, pad), (0, 0)))

    node_lat = _enc_node(node_features, image_feature, params["enc_node"])
    edge_lat = _enc_edge(ef_pad, params["enc_edge"])

    for i, blk in enumerate(params["blocks"]):
        srows, rrows = _sc_gather(node_lat, ix_s3, ix_r3)
        outs = _edge_tc(srows, rrows, edge_lat, blk, want_resid=(i == 0))
        wrows, expl = outs[0], outs[1]
        if i == 0:
            edge_lat = outs[2]
        a0, a1, ss0, ss1 = _sc_scatter(
            wrows, expl.reshape(NW, CH_PER_W, CHUNK), ix_r3)
        node_lat = _node_tc(node_lat, a0, a1,
                            ss0.reshape(N_SP, 1), ss1.reshape(N_SP, 1),
                            blk["node"])

    dec = _dec_tc(node_lat, params["dec"])
    return dec[:, :3].reshape(1, N_NODES, 3)


# speedup vs baseline: 3.9887x; 1.0125x over previous
import functools

import jax
import jax.numpy as jnp
from jax import lax
from jax.experimental import pallas as pl
from jax.experimental.pallas import tpu as pltpu
from jax.experimental.pallas import tpu_sc as plsc

N_NODES = 10000
N_SP = 10240
N_EDGES = 160000
NW = 32
CHUNK = 128
CH_PER_W = 40
E_PER_W = CHUNK * CH_PER_W
E_PAD = NW * E_PER_W
EB = 1280
NB = 2000
ZROWS = N_SP // 16
F32 = jnp.float32


def _ln_rows(x, g, beta):
    mu = jnp.mean(x, axis=-1, keepdims=True)
    var = jnp.mean((x - mu) ** 2, axis=-1, keepdims=True)
    return (x - mu) / jnp.sqrt(var + 1e-5) * g + beta


def _full(shape):
    nd = len(shape)
    return pl.BlockSpec(shape, lambda i: (0,) * nd)


def _enc_node(nf, img, p):
    (W1, b1), (W2, b2) = p["layers"]
    g, beta = p["ln"]
    W1a, W1b = W1[:128], W1[128:]

    def body(nf_ref, img_ref, w1a, w1b, b1_, w2, b2_, g_, be_, o_ref):
        h = nf_ref[...] @ w1a[...] + img_ref[...] @ w1b[...] + b1_[...]
        h = jnp.maximum(h, 0.0)
        y = h @ w2[...] + b2_[...]
        o_ref[...] = _ln_rows(y, g_[...], be_[...])

    return pl.pallas_call(
        body,
        grid=(N_NODES // NB,),
        in_specs=[
            pl.BlockSpec((NB, 128), lambda i: (i, 0)),
            _full((1, 512)), _full((128, 256)), _full((512, 256)),
            _full((256,)), _full((256, 128)), _full((128,)),
            _full((128,)), _full((128,)),
        ],
        out_specs=pl.BlockSpec((NB, 128), lambda i: (i, 0)),
        out_shape=jax.ShapeDtypeStruct((N_NODES, 128), F32),
    )(nf, img, W1a, W1b, b1, W2, b2, g, beta)


def _enc_edge(ef, p):
    (W1, b1), (W2, b2) = p["layers"]
    g, beta = p["ln"]

    def body(ef_ref, w1, b1_, w2, b2_, g_, be_, o_ref):
        h = ef_ref[...] @ w1[...] + b1_[...]
        h = jnp.maximum(h, 0.0)
        y = h @ w2[...] + b2_[...]
        o_ref[...] = _ln_rows(y, g_[...], be_[...])

    return pl.pallas_call(
        body,
        grid=(E_PAD // EB,),
        in_specs=[
            pl.BlockSpec((EB, 16), lambda i: (i, 0)),
            _full((16, 256)), _full((256,)), _full((256, 128)),
            _full((128,)), _full((128,)), _full((128,)),
        ],
        out_specs=pl.BlockSpec((EB, 128), lambda i: (i, 0)),
        out_shape=jax.ShapeDtypeStruct((E_PAD, 128), F32),
    )(ef, W1, b1, W2, b2, g, beta)


def _edge_tc(srows, rrows, elat, blk, want_resid):
    (W1, b1), (W2, b2), (W3, b3) = blk["edge"]["layers"]
    g, beta = blk["edge"]["ln"]
    aW, ab = blk["att"]
    W1s, W1r, W1e = W1[:128], W1[128:256], W1[256:]
    aWr = aW.reshape(1, 128)
    ab2 = ab.reshape(1, 1)

    def body(s_ref, r_ref, e_ref, w1s, w1r, w1e, b1_, w2, b2_, w3, b3_,
             g_, be_, aw, ab_, w_ref, x_ref, *res):
        h = (s_ref[...] @ w1s[...] + r_ref[...] @ w1r[...]
             + e_ref[...] @ w1e[...] + b1_[...])
        h = jnp.maximum(h, 0.0)
        h = jnp.maximum(h @ w2[...] + b2_[...], 0.0)
        ne = _ln_rows(h @ w3[...] + b3_[...], g_[...], be_[...])
        lg = jnp.sum(ne * aw[...], axis=-1, keepdims=True) + ab_[...]
        lg = jnp.where(lg >= 0, lg, 0.2 * lg)
        ex = jnp.exp(lg)
        w_ref[...] = ne * ex
        x_ref[...] = ex
        if want_resid:
            res[0][...] = ne + e_ref[...]

    out_shape = [jax.ShapeDtypeStruct((E_PAD, 128), F32),
                 jax.ShapeDtypeStruct((E_PAD, 1), F32)]
    out_specs = [pl.BlockSpec((EB, 128), lambda i: (i, 0)),
                 pl.BlockSpec((EB, 1), lambda i: (i, 0))]
    if want_resid:
        out_shape.append(jax.ShapeDtypeStruct((E_PAD, 128), F32))
        out_specs.append(pl.BlockSpec((EB, 128), lambda i: (i, 0)))

    return pl.pallas_call(
        body,
        grid=(E_PAD // EB,),
        in_specs=[
            pl.BlockSpec((EB, 128), lambda i: (i, 0)),
            pl.BlockSpec((EB, 128), lambda i: (i, 0)),
            pl.BlockSpec((EB, 128), lambda i: (i, 0)),
            _full((128, 128)), _full((128, 128)), _full((128, 128)),
            _full((128,)), _full((128, 128)), _full((128,)),
            _full((128, 128)), _full((128,)),
            _full((128,)), _full((128,)), _full((1, 128)), _full((1, 1)),
        ],
        out_specs=out_specs,
        out_shape=out_shape,
    )(srows, rrows, elat, W1s, W1r, W1e, b1, W2, b2, W3, b3, g, beta,
      aWr, ab2)


def _node_tc(nlat, a0, a1, s0, s1, p):
    (W1, b1), (W2, b2), (W3, b3) = p["layers"]
    g, beta = p["ln"]
    W1n, W1a = W1[:128], W1[128:]

    def body(n_ref, a0_, a1_, s0_, s1_, w1n, w1a, b1_, w2, b2_, w3, b3_,
             g_, be_, o_ref):
        n = n_ref[...]
        agg = (a0_[...] + a1_[...]) / (s0_[...] + s1_[...] + 1e-16)
        h = jnp.maximum(n @ w1n[...] + agg @ w1a[...] + b1_[...], 0.0)
        h = jnp.maximum(h @ w2[...] + b2_[...], 0.0)
        y = _ln_rows(h @ w3[...] + b3_[...], g_[...], be_[...])
        o_ref[...] = y + n

    return pl.pallas_call(
        body,
        grid=(N_NODES // NB,),
        in_specs=[
            pl.BlockSpec((NB, 128), lambda i: (i, 0)),
            pl.BlockSpec((NB, 128), lambda i: (i, 0)),
            pl.BlockSpec((NB, 128), lambda i: (i, 0)),
            pl.BlockSpec((NB, 1), lambda i: (i, 0)),
            pl.BlockSpec((NB, 1), lambda i: (i, 0)),
            _full((128, 128)), _full((128, 128)), _full((128,)),
            _full((128, 128)), _full((128,)), _full((128, 128)),
            _full((128,)), _full((128,)), _full((128,)),
        ],
        out_specs=pl.BlockSpec((NB, 128), lambda i: (i, 0)),
        out_shape=jax.ShapeDtypeStruct((N_NODES, 128), F32),
    )(nlat, a0, a1, s0, s1, W1n, W1a, b1, W2, b2, W3, b3, g, beta)


def _dec_tc(nlat, p):
    (W1, b1), (W2, b2), (W3, b3) = p["layers"]
    W3p = jnp.zeros((128, 128), F32).at[:, :W3.shape[1]].set(W3)
    b3p = jnp.zeros((128,), F32).at[:b3.shape[0]].set(b3)

    def body(n_ref, w1, b1_, w2, b2_, w3, b3_, o_ref):
        h = jnp.maximum(n_ref[...] @ w1[...] + b1_[...], 0.0)
        h = jnp.maximum(h @ w2[...] + b2_[...], 0.0)
        o_ref[...] = h @ w3[...] + b3_[...]

    return pl.pallas_call(
        body,
        grid=(N_NODES // NB,),
        in_specs=[
            pl.BlockSpec((NB, 128), lambda i: (i, 0)),
            _full((128, 128)), _full((128,)), _full((128, 128)),
            _full((128,)), _full((128, 128)), _full((128,)),
        ],
        out_specs=pl.BlockSpec((NB, 128), lambda i: (i, 0)),
        out_shape=jax.ShapeDtypeStruct((N_NODES, 128), F32),
    )(nlat, W1, b1, W2, b2, W3p, b3p)


def _sc_gather(table, ix_s3, ix_r3):
    mesh = plsc.VectorSubcoreMesh(core_axis_name="c", subcore_axis_name="s")

    @functools.partial(
        pl.kernel,
        out_type=(jax.ShapeDtypeStruct((E_PAD, 128), F32),
                  jax.ShapeDtypeStruct((E_PAD, 128), F32)),
        mesh=mesh,
        scratch_types=[
            pltpu.VMEM((CH_PER_W, CHUNK), jnp.int32),
            pltpu.VMEM((CH_PER_W, CHUNK), jnp.int32),
            pltpu.VMEM((CHUNK, 128), F32),
            pltpu.VMEM((CHUNK, 128), F32),
            pltpu.VMEM((CHUNK, 128), F32),
            pltpu.VMEM((CHUNK, 128), F32),
            [pltpu.SemaphoreType.DMA] * 8,
        ],
    )
    def k(table_hbm, ixs_hbm, ixr_hbm, os_hbm, or_hbm,
          ixs_v, ixr_v, bsA, bsB, brA, brB, sems):
        gsA, gsB, grA, grB, osA, osB, orA, orB = sems
        wid = lax.axis_index("s") * 2 + lax.axis_index("c")
        pltpu.sync_copy(ixs_hbm.at[wid], ixs_v)
        pltpu.sync_copy(ixr_hbm.at[wid], ixr_v)
        base = wid * E_PER_W

        def g_start(ix_v, buf, sem, c):
            pltpu.async_copy(table_hbm.at[ix_v.at[c]], buf, sem)

        def g_wait(ix_v, buf, sem, c):
            pltpu.make_async_copy(table_hbm.at[ix_v.at[c]], buf, sem).wait()

        def o_start(buf, out_hbm, sem, c):
            pltpu.async_copy(
                buf, out_hbm.at[pl.ds(base + c * CHUNK, CHUNK)], sem)

        def o_wait(buf, out_hbm, sem, c):
            pltpu.make_async_copy(
                buf, out_hbm.at[pl.ds(base + c * CHUNK, CHUNK)], sem).wait()

        g_start(ixs_v, bsA, gsA, 0)
        g_start(ixr_v, brA, grA, 0)
        g_start(ixs_v, bsB, gsB, 1)
        g_start(ixr_v, brB, grB, 1)

        def body(j, carry):
            c0 = 2 * j
            c1 = c0 + 1
            g_wait(ixs_v, bsA, gsA, c0)
            o_start(bsA, os_hbm, osA, c0)
            g_wait(ixr_v, brA, grA, c0)
            o_start(brA, or_hbm, orA, c0)
            g_wait(ixs_v, bsB, gsB, c1)
            o_start(bsB, os_hbm, osB, c1)
            g_wait(ixr_v, brB, grB, c1)
            o_start(brB, or_hbm, orB, c1)
            o_wait(bsA, os_hbm, osA, c0)
            g_start(ixs_v, bsA, gsA, c0 + 2)
            o_wait(brA, or_hbm, orA, c0)
            g_start(ixr_v, brA, grA, c0 + 2)
            o_wait(bsB, os_hbm, osB, c1)
            g_start(ixs_v, bsB, gsB, c1 + 2)
            o_wait(brB, or_hbm, orB, c1)
            g_start(ixr_v, brB, grB, c1 + 2)
            return carry

        lax.fori_loop(0, CH_PER_W // 2 - 1, body, 0)
        c0 = CH_PER_W - 2
        c1 = CH_PER_W - 1
        g_wait(ixs_v, bsA, gsA, c0)
        o_start(bsA, os_hbm, osA, c0)
        g_wait(ixr_v, brA, grA, c0)
        o_start(brA, or_hbm, orA, c0)
        g_wait(ixs_v, bsB, gsB, c1)
        o_start(bsB, os_hbm, osB, c1)
        g_wait(ixr_v, brB, grB, c1)
        o_start(brB, or_hbm, orB, c1)
        o_wait(bsA, os_hbm, osA, c0)
        o_wait(brA, or_hbm, orA, c0)
        o_wait(bsB, os_hbm, osB, c1)
        o_wait(brB, or_hbm, orB, c1)

    return k(table, ix_s3, ix_r3)


def _sc_scatter(wrows, expl3, ix_r3):
    mesh = plsc.VectorSubcoreMesh(core_axis_name="c", subcore_axis_name="s")

    @functools.partial(
        pl.kernel,
        out_type=(jax.ShapeDtypeStruct((N_SP, 128), F32),
                  jax.ShapeDtypeStruct((N_SP, 128), F32),
                  jax.ShapeDtypeStruct((N_SP,), F32),
                  jax.ShapeDtypeStruct((N_SP,), F32)),
        mesh=mesh,
        scratch_types=[
            pltpu.VMEM_SHARED((N_SP, 128), F32),
            pltpu.VMEM_SHARED((N_SP,), F32),
            pltpu.VMEM((CH_PER_W, CHUNK), jnp.int32),
            pltpu.VMEM((CH_PER_W, CHUNK), F32),
            pltpu.VMEM((CHUNK, 128), F32),
            pltpu.VMEM((CHUNK, 128), F32),
            [pltpu.SemaphoreType.DMA] * 4,
        ],
    )
    def k(w_hbm, ex_hbm, ix_hbm, a0_hbm, a1_hbm, s0_hbm, s1_hbm,
          spa, sps, ix_v, ex_v, bufA, bufB, sems):
        lA, lB, sA, sB = sems
        cid = lax.axis_index("c")
        sid = lax.axis_index("s")
        wid = sid * 2 + cid

        def zrow(i, carry):
            for j in range(8):
                bufA[i, pl.ds(j * 16, 16)] = jnp.zeros((16,), F32)
            return carry

        lax.fori_loop(0, CHUNK, zrow, 0)
        z0 = sid * ZROWS
        for kk in range(ZROWS // CHUNK):
            pltpu.sync_copy(bufA, spa.at[pl.ds(z0 + kk * CHUNK, CHUNK)])
            pltpu.sync_copy(bufA.at[0], sps.at[pl.ds(z0 + kk * CHUNK, CHUNK)])
        plsc.subcore_barrier()

        pltpu.sync_copy(ix_hbm.at[wid], ix_v)
        pltpu.sync_copy(ex_hbm.at[wid], ex_v)
        base = wid * E_PER_W

        def l_start(buf, sem, c):
            pltpu.async_copy(
                w_hbm.at[pl.ds(base + c * CHUNK, CHUNK)], buf, sem)

        def l_wait(buf, sem, c):
            pltpu.make_async_copy(
                w_hbm.at[pl.ds(base + c * CHUNK, CHUNK)], buf, sem).wait()

        def s_start(buf, sem, c):
            pltpu.async_copy(buf, spa.at[ix_v.at[c]], sem, add=True)
            pltpu.async_copy(ex_v.at[c], sps.at[ix_v.at[c]], sem, add=True)

        def s_wait(buf, sem, c):
            pltpu.make_async_copy(buf, spa.at[ix_v.at[c]], sem).wait()
            pltpu.make_async_copy(ex_v.at[c], sps.at[ix_v.at[c]], sem).wait()

        l_start(bufA, lA, 0)
        l_start(bufB, lB, 1)

        def step(j, carry):
            c0 = 2 * j
            c1 = c0 + 1
            l_wait(bufA, lA, c0)
            s_start(bufA, sA, c0)
            l_wait(bufB, lB, c1)
            s_start(bufB, sB, c1)
            s_wait(bufA, sA, c0)
            l_start(bufA, lA, c0 + 2)
            s_wait(bufB, sB, c1)
            l_start(bufB, lB, c1 + 2)
            return carry

        lax.fori_loop(0, CH_PER_W // 2 - 1, step, 0)
        c0 = CH_PER_W - 2
        c1 = CH_PER_W - 1
        l_wait(bufA, lA, c0)
        s_start(bufA, sA, c0)
        l_wait(bufB, lB, c1)
        s_start(bufB, sB, c1)
        s_wait(bufA, sA, c0)
        s_wait(bufB, sB, c1)
        plsc.subcore_barrier()

        @pl.when(cid == 0)
        def _():
            pltpu.sync_copy(spa.at[pl.ds(z0, ZROWS)], a0_hbm.at[pl.ds(z0, ZROWS)])
            pltpu.sync_copy(sps.at[pl.ds(z0, ZROWS)], s0_hbm.at[pl.ds(z0, ZROWS)])

        @pl.when(cid == 1)
        def _():
            pltpu.sync_copy(spa.at[pl.ds(z0, ZROWS)], a1_hbm.at[pl.ds(z0, ZROWS)])
            pltpu.sync_copy(sps.at[pl.ds(z0, ZROWS)], s1_hbm.at[pl.ds(z0, ZROWS)])

    return k(wrows, expl3, ix_r3)


def kernel(node_features, edge_features, senders, receivers, image_feature,
           params):
    pad = E_PAD - N_EDGES
    s_pad = jnp.concatenate([senders, jnp.zeros((pad,), jnp.int32)])
    r_pad = jnp.concatenate([receivers, jnp.full((pad,), N_NODES, jnp.int32)])
    ix_s3 = s_pad.reshape(NW, CH_PER_W, CHUNK)
    ix_r3 = r_pad.reshape(NW, CH_PER_W, CHUNK)
    ef_pad = jnp.pad(edge_features, ((0, pad), (0, 0)))

    node_lat = _enc_node(node_features, image_feature, params["enc_node"])
    edge_lat = _enc_edge(ef_pad, params["enc_edge"])

    for i, blk in enumerate(params["blocks"]):
        srows, rrows = _sc_gather(node_lat, ix_s3, ix_r3)
        outs = _edge_tc(srows, rrows, edge_lat, blk, want_resid=(i == 0))
        wrows, expl = outs[0], outs[1]
        if i == 0:
            edge_lat = outs[2]
        a0, a1, ss0, ss1 = _sc_scatter(
            wrows, expl.reshape(NW, CH_PER_W, CHUNK), ix_r3)
        node_lat = _node_tc(node_lat, a0, a1,
                            ss0.reshape(N_SP, 1), ss1.reshape(N_SP, 1),
                            blk["node"])

    dec = _dec_tc(node_lat, params["dec"])
    return dec[:, :3].reshape(1, N_NODES, 3)


# speedup vs baseline: 4.2876x; 1.0749x over previous
import functools

import jax
import jax.numpy as jnp
from jax import lax
from jax.experimental import pallas as pl
from jax.experimental.pallas import tpu as pltpu
from jax.experimental.pallas import tpu_sc as plsc

N_NODES = 10000
N_SP = 10240
N_EDGES = 160000
NW = 32
CHUNK = 128
CH_PER_W = 40
E_PER_W = CHUNK * CH_PER_W
E_PAD = NW * E_PER_W
EB = 1280
NB = 2000
ZROWS = N_SP // 16
SID_CH = 80
GK0 = 56
F32 = jnp.float32


def _ln_rows(x, g, beta):
    mu = jnp.mean(x, axis=-1, keepdims=True)
    var = jnp.mean((x - mu) ** 2, axis=-1, keepdims=True)
    return (x - mu) / jnp.sqrt(var + 1e-5) * g + beta


def _full(shape):
    nd = len(shape)
    return pl.BlockSpec(shape, lambda i: (0,) * nd)


def _enc_node(nf, img, p):
    (W1, b1), (W2, b2) = p["layers"]
    g, beta = p["ln"]
    W1a, W1b = W1[:128], W1[128:]

    def body(nf_ref, img_ref, w1a, w1b, b1_, w2, b2_, g_, be_, o_ref):
        h = nf_ref[...] @ w1a[...] + img_ref[...] @ w1b[...] + b1_[...]
        h = jnp.maximum(h, 0.0)
        y = h @ w2[...] + b2_[...]
        o_ref[...] = _ln_rows(y, g_[...], be_[...])

    return pl.pallas_call(
        body,
        grid=(N_NODES // NB,),
        in_specs=[
            pl.BlockSpec((NB, 128), lambda i: (i, 0)),
            _full((1, 512)), _full((128, 256)), _full((512, 256)),
            _full((256,)), _full((256, 128)), _full((128,)),
            _full((128,)), _full((128,)),
        ],
        out_specs=pl.BlockSpec((NB, 128), lambda i: (i, 0)),
        out_shape=jax.ShapeDtypeStruct((N_NODES, 128), F32),
    )(nf, img, W1a, W1b, b1, W2, b2, g, beta)


def _enc_edge(ef, p):
    (W1, b1), (W2, b2) = p["layers"]
    g, beta = p["ln"]

    def body(ef_ref, w1, b1_, w2, b2_, g_, be_, o_ref):
        h = ef_ref[...] @ w1[...] + b1_[...]
        h = jnp.maximum(h, 0.0)
        y = h @ w2[...] + b2_[...]
        o_ref[...] = _ln_rows(y, g_[...], be_[...])

    last = N_EDGES // EB - 1
    return pl.pallas_call(
        body,
        grid=(E_PAD // EB,),
        in_specs=[
            pl.BlockSpec((EB, 16), lambda i: (jnp.minimum(i, last), 0)),
            _full((16, 256)), _full((256,)), _full((256, 128)),
            _full((128,)), _full((128,)), _full((128,)),
        ],
        out_specs=pl.BlockSpec((EB, 128), lambda i: (i, 0)),
        out_shape=jax.ShapeDtypeStruct((E_PAD, 128), F32),
    )(ef, W1, b1, W2, b2, g, beta)


def _edge_tc(srows, rrows, elat, blk, want_resid):
    (W1, b1), (W2, b2), (W3, b3) = blk["edge"]["layers"]
    g, beta = blk["edge"]["ln"]
    aW, ab = blk["att"]
    W1s, W1r, W1e = W1[:128], W1[128:256], W1[256:]
    aWr = aW.reshape(1, 128)
    ab2 = ab.reshape(1, 1)

    def body(s_ref, r_ref, e_ref, w1s, w1r, w1e, b1_, w2, b2_, w3, b3_,
             g_, be_, aw, ab_, w_ref, x_ref, *res):
        h = (s_ref[...] @ w1s[...] + r_ref[...] @ w1r[...]
             + e_ref[...] @ w1e[...] + b1_[...])
        h = jnp.maximum(h, 0.0)
        h = jnp.maximum(h @ w2[...] + b2_[...], 0.0)
        ne = _ln_rows(h @ w3[...] + b3_[...], g_[...], be_[...])
        lg = jnp.sum(ne * aw[...], axis=-1, keepdims=True) + ab_[...]
        lg = jnp.where(lg >= 0, lg, 0.2 * lg)
        ex = jnp.exp(lg)
        w_ref[...] = ne * ex
        x_ref[...] = ex
        if want_resid:
            res[0][...] = ne + e_ref[...]

    out_shape = [jax.ShapeDtypeStruct((E_PAD, 128), F32),
                 jax.ShapeDtypeStruct((E_PAD, 1), F32)]
    out_specs = [pl.BlockSpec((EB, 128), lambda i: (i, 0)),
                 pl.BlockSpec((EB, 1), lambda i: (i, 0))]
    if want_resid:
        out_shape.append(jax.ShapeDtypeStruct((E_PAD, 128), F32))
        out_specs.append(pl.BlockSpec((EB, 128), lambda i: (i, 0)))

    return pl.pallas_call(
        body,
        grid=(E_PAD // EB,),
        in_specs=[
            pl.BlockSpec((EB, 128), lambda i: (i, 0)),
            pl.BlockSpec((EB, 128), lambda i: (i, 0)),
            pl.BlockSpec((EB, 128), lambda i: (i, 0)),
            _full((128, 128)), _full((128, 128)), _full((128, 128)),
            _full((128,)), _full((128, 128)), _full((128,)),
            _full((128, 128)), _full((128,)),
            _full((128,)), _full((128,)), _full((1, 128)), _full((1, 1)),
        ],
        out_specs=out_specs,
        out_shape=out_shape,
    )(srows, rrows, elat, W1s, W1r, W1e, b1, W2, b2, W3, b3, g, beta,
      aWr, ab2)


def _node_tc(nlat, a0, a1, s0, s1, p):
    (W1, b1), (W2, b2), (W3, b3) = p["layers"]
    g, beta = p["ln"]
    W1n, W1a = W1[:128], W1[128:]

    def body(n_ref, a0_, a1_, s0_, s1_, w1n, w1a, b1_, w2, b2_, w3, b3_,
             g_, be_, o_ref):
        n = n_ref[...]
        agg = (a0_[...] + a1_[...]) / (s0_[...] + s1_[...] + 1e-16)
        h = jnp.maximum(n @ w1n[...] + agg @ w1a[...] + b1_[...], 0.0)
        h = jnp.maximum(h @ w2[...] + b2_[...], 0.0)
        y = _ln_rows(h @ w3[...] + b3_[...], g_[...], be_[...])
        o_ref[...] = y + n

    return pl.pallas_call(
        body,
        grid=(N_NODES // NB,),
        in_specs=[
            pl.BlockSpec((NB, 128), lambda i: (i, 0)),
            pl.BlockSpec((NB, 128), lambda i: (i, 0)),
            pl.BlockSpec((NB, 128), lambda i: (i, 0)),
            pl.BlockSpec((NB, 1), lambda i: (i, 0)),
            pl.BlockSpec((NB, 1), lambda i: (i, 0)),
            _full((128, 128)), _full((128, 128)), _full((128,)),
            _full((128, 128)), _full((128,)), _full((128, 128)),
            _full((128,)), _full((128,)), _full((128,)),
        ],
        out_specs=pl.BlockSpec((NB, 128), lambda i: (i, 0)),
        out_shape=jax.ShapeDtypeStruct((N_NODES, 128), F32),
    )(nlat, a0, a1, s0, s1, W1n, W1a, b1, W2, b2, W3, b3, g, beta)


def _dec_tc(nlat, p):
    (W1, b1), (W2, b2), (W3, b3) = p["layers"]
    W3p = jnp.zeros((128, 128), F32).at[:, :W3.shape[1]].set(W3)
    b3p = jnp.zeros((128,), F32).at[:b3.shape[0]].set(b3)

    def body(n_ref, w1, b1_, w2, b2_, w3, b3_, o_ref):
        h = jnp.maximum(n_ref[...] @ w1[...] + b1_[...], 0.0)
        h = jnp.maximum(h @ w2[...] + b2_[...], 0.0)
        o_ref[...] = h @ w3[...] + b3_[...]

    return pl.pallas_call(
        body,
        grid=(N_NODES // NB,),
        in_specs=[
            pl.BlockSpec((NB, 128), lambda i: (i, 0)),
            _full((128, 128)), _full((128,)), _full((128, 128)),
            _full((128,)), _full((128, 128)), _full((128,)),
        ],
        out_specs=pl.BlockSpec((NB, 128), lambda i: (i, 0)),
        out_shape=jax.ShapeDtypeStruct((N_NODES, 128), F32),
    )(nlat, W1, b1, W2, b2, W3p, b3p)


def _sc_gather(table, ix_s3, ix_r3):
    mesh = plsc.VectorSubcoreMesh(core_axis_name="c", subcore_axis_name="s")

    @functools.partial(
        pl.kernel,
        out_type=(jax.ShapeDtypeStruct((E_PAD, 128), F32),
                  jax.ShapeDtypeStruct((E_PAD, 128), F32)),
        mesh=mesh,
        scratch_types=[
            pltpu.VMEM((SID_CH, CHUNK), jnp.int32),
            pltpu.VMEM((SID_CH, CHUNK), jnp.int32),
            pltpu.VMEM((CHUNK, 128), F32),
            pltpu.VMEM((CHUNK, 128), F32),
            pltpu.VMEM((CHUNK, 128), F32),
            pltpu.VMEM((CHUNK, 128), F32),
            [pltpu.SemaphoreType.DMA] * 8,
        ],
    )
    def k(table_hbm, ixs_hbm, ixr_hbm, os_hbm, or_hbm,
          ixs_v, ixr_v, bsA, bsB, brA, brB, sems):
        gsA, gsB, grA, grB, osA, osB, orA, orB = sems
        cid = lax.axis_index("c")
        sid = lax.axis_index("s")
        pltpu.sync_copy(ixs_hbm.at[sid], ixs_v)
        pltpu.sync_copy(ixr_hbm.at[sid], ixr_v)
        loc0 = jnp.where(cid == 0, 0, GK0)
        nch = jnp.where(cid == 0, GK0, SID_CH - GK0)
        base = (sid * SID_CH + loc0) * CHUNK

        def g_start(ix_v, buf, sem, c):
            pltpu.async_copy(table_hbm.at[ix_v.at[loc0 + c]], buf, sem)

        def g_wait(ix_v, buf, sem, c):
            pltpu.make_async_copy(
                table_hbm.at[ix_v.at[loc0 + c]], buf, sem).wait()

        def o_start(buf, out_hbm, sem, c):
            pltpu.async_copy(
                buf, out_hbm.at[pl.ds(base + c * CHUNK, CHUNK)], sem)

        def o_wait(buf, out_hbm, sem, c):
            pltpu.make_async_copy(
                buf, out_hbm.at[pl.ds(base + c * CHUNK, CHUNK)], sem).wait()

        g_start(ixs_v, bsA, gsA, 0)
        g_start(ixr_v, brA, grA, 0)
        g_start(ixs_v, bsB, gsB, 1)
        g_start(ixr_v, brB, grB, 1)

        def body(j, carry):
            c0 = 2 * j
            c1 = c0 + 1
            g_wait(ixs_v, bsA, gsA, c0)
            o_start(bsA, os_hbm, osA, c0)
            g_wait(ixr_v, brA, grA, c0)
            o_start(brA, or_hbm, orA, c0)
            g_wait(ixs_v, bsB, gsB, c1)
            o_start(bsB, os_hbm, osB, c1)
            g_wait(ixr_v, brB, grB, c1)
            o_start(brB, or_hbm, orB, c1)
            o_wait(bsA, os_hbm, osA, c0)
            g_start(ixs_v, bsA, gsA, c0 + 2)
            o_wait(brA, or_hbm, orA, c0)
            g_start(ixr_v, brA, grA, c0 + 2)
            o_wait(bsB, os_hbm, osB, c1)
            g_start(ixs_v, bsB, gsB, c1 + 2)
            o_wait(brB, or_hbm, orB, c1)
            g_start(ixr_v, brB, grB, c1 + 2)
            return carry

        lax.fori_loop(0, nch // 2 - 1, body, 0)
        c0 = nch - 2
        c1 = nch - 1
        g_wait(ixs_v, bsA, gsA, c0)
        o_start(bsA, os_hbm, osA, c0)
        g_wait(ixr_v, brA, grA, c0)
        o_start(brA, or_hbm, orA, c0)
        g_wait(ixs_v, bsB, gsB, c1)
        o_start(bsB, os_hbm, osB, c1)
        g_wait(ixr_v, brB, grB, c1)
        o_start(brB, or_hbm, orB, c1)
        o_wait(bsA, os_hbm, osA, c0)
        o_wait(brA, or_hbm, orA, c0)
        o_wait(bsB, os_hbm, osB, c1)
        o_wait(brB, or_hbm, orB, c1)

    return k(table, ix_s3, ix_r3)


def _sc_scatter(wrows, expl3, ix_r3):
    mesh = plsc.VectorSubcoreMesh(core_axis_name="c", subcore_axis_name="s")

    @functools.partial(
        pl.kernel,
        out_type=(jax.ShapeDtypeStruct((N_SP, 128), F32),
                  jax.ShapeDtypeStruct((N_SP, 128), F32),
                  jax.ShapeDtypeStruct((N_SP,), F32),
                  jax.ShapeDtypeStruct((N_SP,), F32)),
        mesh=mesh,
        scratch_types=[
            pltpu.VMEM_SHARED((N_SP, 128), F32),
            pltpu.VMEM_SHARED((N_SP,), F32),
            pltpu.VMEM((CH_PER_W, CHUNK), jnp.int32),
            pltpu.VMEM((E_PER_W,), F32),
            pltpu.VMEM((CHUNK, 128), F32),
            pltpu.VMEM((CHUNK, 128), F32),
            [pltpu.SemaphoreType.DMA] * 4,
        ],
    )
    def k(w_hbm, ex_hbm, ix_hbm, a0_hbm, a1_hbm, s0_hbm, s1_hbm,
          spa, sps, ix_v, ex_v, bufA, bufB, sems):
        lA, lB, sA, sB = sems
        cid = lax.axis_index("c")
        sid = lax.axis_index("s")

        def zrow(i, carry):
            for j in range(8):
                bufA[i, pl.ds(j * 16, 16)] = jnp.zeros((16,), F32)
            return carry

        lax.fori_loop(0, CHUNK, zrow, 0)
        z0 = sid * ZROWS
        for kk in range(ZROWS // CHUNK):
            pltpu.sync_copy(bufA, spa.at[pl.ds(z0 + kk * CHUNK, CHUNK)])
            pltpu.sync_copy(bufA.at[0], sps.at[pl.ds(z0 + kk * CHUNK, CHUNK)])
        plsc.subcore_barrier()

        pltpu.sync_copy(ix_hbm.at[sid, pl.ds(cid * CH_PER_W, CH_PER_W)], ix_v)
        base = (sid * SID_CH + cid * CH_PER_W) * CHUNK
        pltpu.sync_copy(ex_hbm.at[pl.ds(base, E_PER_W)], ex_v)

        def l_start(buf, sem, c):
            pltpu.async_copy(
                w_hbm.at[pl.ds(base + c * CHUNK, CHUNK)], buf, sem)

        def l_wait(buf, sem, c):
            pltpu.make_async_copy(
                w_hbm.at[pl.ds(base + c * CHUNK, CHUNK)], buf, sem).wait()

        def s_start(buf, sem, c):
            pltpu.async_copy(buf, spa.at[ix_v.at[c]], sem, add=True)
            pltpu.async_copy(
                ex_v.at[pl.ds(c * CHUNK, CHUNK)], sps.at[ix_v.at[c]],
                sem, add=True)

        def s_wait(buf, sem, c):
            pltpu.make_async_copy(buf, spa.at[ix_v.at[c]], sem).wait()
            pltpu.make_async_copy(
                ex_v.at[pl.ds(c * CHUNK, CHUNK)], sps.at[ix_v.at[c]],
                sem).wait()

        l_start(bufA, lA, 0)
        l_start(bufB, lB, 1)

        def step(j, carry):
            c0 = 2 * j
            c1 = c0 + 1
            l_wait(bufA, lA, c0)
            s_start(bufA, sA, c0)
            l_wait(bufB, lB, c1)
            s_start(bufB, sB, c1)
            s_wait(bufA, sA, c0)
            l_start(bufA, lA, c0 + 2)
            s_wait(bufB, sB, c1)
            l_start(bufB, lB, c1 + 2)
            return carry

        lax.fori_loop(0, CH_PER_W // 2 - 1, step, 0)
        c0 = CH_PER_W - 2
        c1 = CH_PER_W - 1
        l_wait(bufA, lA, c0)
        s_start(bufA, sA, c0)
        l_wait(bufB, lB, c1)
        s_start(bufB, sB, c1)
        s_wait(bufA, sA, c0)
        s_wait(bufB, sB, c1)
        plsc.subcore_barrier()

        @pl.when(cid == 0)
        def _():
            pltpu.sync_copy(spa.at[pl.ds(z0, ZROWS)], a0_hbm.at[pl.ds(z0, ZROWS)])
            pltpu.sync_copy(sps.at[pl.ds(z0, ZROWS)], s0_hbm.at[pl.ds(z0, ZROWS)])

        @pl.when(cid == 1)
        def _():
            pltpu.sync_copy(spa.at[pl.ds(z0, ZROWS)], a1_hbm.at[pl.ds(z0, ZROWS)])
            pltpu.sync_copy(sps.at[pl.ds(z0, ZROWS)], s1_hbm.at[pl.ds(z0, ZROWS)])

    return k(wrows, expl3, ix_r3)


def kernel(node_features, edge_features, senders, receivers, image_feature,
           params):
    pad = E_PAD - N_EDGES
    s_pad = jnp.concatenate([senders, jnp.zeros((pad,), jnp.int32)])
    r_pad = jnp.concatenate([receivers, jnp.full((pad,), N_NODES, jnp.int32)])
    ix_s3 = s_pad.reshape(16, SID_CH, CHUNK)
    ix_r3 = r_pad.reshape(16, SID_CH, CHUNK)

    node_lat = _enc_node(node_features, image_feature, params["enc_node"])
    edge_lat = _enc_edge(edge_features, params["enc_edge"])

    for i, blk in enumerate(params["blocks"]):
        srows, rrows = _sc_gather(node_lat, ix_s3, ix_r3)
        outs = _edge_tc(srows, rrows, edge_lat, blk, want_resid=(i == 0))
        wrows, expl = outs[0], outs[1]
        if i == 0:
            edge_lat = outs[2]
        a0, a1, ss0, ss1 = _sc_scatter(
            wrows, expl.reshape(E_PAD), ix_r3)
        node_lat = _node_tc(node_lat, a0, a1,
                            ss0.reshape(N_SP, 1), ss1.reshape(N_SP, 1),
                            blk["node"])

    dec = _dec_tc(node_lat, params["dec"])
    return dec[:, :3].reshape(1, N_NODES, 3)


# speedup vs baseline: 4.6056x; 1.0742x over previous
import functools

import jax
import jax.numpy as jnp
from jax import lax
from jax.experimental import pallas as pl
from jax.experimental.pallas import tpu as pltpu
from jax.experimental.pallas import tpu_sc as plsc

N_NODES = 10000
N_SP = 10240
N_EDGES = 160000
CHUNK = 128
CH_PER_W = 40
E_PER_W = CHUNK * CH_PER_W
E_PAD = 32 * E_PER_W
EB = 1280
NB = 2000
ZROWS = N_SP // 16
SID_CH = 80
GK0 = 60
F32 = jnp.float32
BF16 = jnp.bfloat16


def _ln_rows(x, g, beta):
    mu = jnp.mean(x, axis=-1, keepdims=True)
    var = jnp.mean((x - mu) ** 2, axis=-1, keepdims=True)
    return (x - mu) / jnp.sqrt(var + 1e-5) * g + beta


def _full(shape):
    nd = len(shape)
    return pl.BlockSpec(shape, lambda i: (0,) * nd)


def _enc_node(nf, img, p):
    (W1, b1), (W2, b2) = p["layers"]
    g, beta = p["ln"]
    W1a, W1b = W1[:128], W1[128:]

    def body(nf_ref, img_ref, w1a, w1b, b1_, w2, b2_, g_, be_, o_ref):
        h = nf_ref[...] @ w1a[...] + img_ref[...] @ w1b[...] + b1_[...]
        h = jnp.maximum(h, 0.0)
        y = h @ w2[...] + b2_[...]
        o_ref[...] = _ln_rows(y, g_[...], be_[...])

    return pl.pallas_call(
        body,
        grid=(N_NODES // NB,),
        in_specs=[
            pl.BlockSpec((NB, 128), lambda i: (i, 0)),
            _full((1, 512)), _full((128, 256)), _full((512, 256)),
            _full((256,)), _full((256, 128)), _full((128,)),
            _full((128,)), _full((128,)),
        ],
        out_specs=pl.BlockSpec((NB, 128), lambda i: (i, 0)),
        out_shape=jax.ShapeDtypeStruct((N_NODES, 128), F32),
    )(nf, img, W1a, W1b, b1, W2, b2, g, beta)


def _enc_edge(ef, p):
    (W1, b1), (W2, b2) = p["layers"]
    g, beta = p["ln"]

    def body(ef_ref, w1, b1_, w2, b2_, g_, be_, o_ref):
        h = ef_ref[...] @ w1[...] + b1_[...]
        h = jnp.maximum(h, 0.0)
        y = h @ w2[...] + b2_[...]
        o_ref[...] = _ln_rows(y, g_[...], be_[...])

    last = N_EDGES // EB - 1
    return pl.pallas_call(
        body,
        grid=(E_PAD // EB,),
        in_specs=[
            pl.BlockSpec((EB, 16), lambda i: (jnp.minimum(i, last), 0)),
            _full((16, 256)), _full((256,)), _full((256, 128)),
            _full((128,)), _full((128,)), _full((128,)),
        ],
        out_specs=pl.BlockSpec((EB, 128), lambda i: (i, 0)),
        out_shape=jax.ShapeDtypeStruct((E_PAD, 128), F32),
    )(ef, W1, b1, W2, b2, g, beta)


def _proj(nlat, blk):
    (W1, b1), _, _ = blk["edge"]["layers"]
    W1s, W1r = W1[:128], W1[128:256]

    def body(n_ref, w1s, w1r, b1_, ps_ref, pr_ref):
        n = n_ref[...]
        ps_ref[...] = n @ w1s[...] + b1_[...]
        pr_ref[...] = n @ w1r[...]

    return pl.pallas_call(
        body,
        grid=(N_NODES // NB,),
        in_specs=[
            pl.BlockSpec((NB, 128), lambda i: (i, 0)),
            _full((128, 128)), _full((128, 128)), _full((128,)),
        ],
        out_specs=[pl.BlockSpec((NB, 128), lambda i: (i, 0)),
                   pl.BlockSpec((NB, 128), lambda i: (i, 0))],
        out_shape=[jax.ShapeDtypeStruct((N_NODES, 128), F32),
                   jax.ShapeDtypeStruct((N_NODES, 128), F32)],
    )(nlat, W1s, W1r, b1)


def _edge_tc(pre, elat, blk, want_resid):
    _, (W2, b2), (W3, b3) = blk["edge"]["layers"]
    (W1, _), _, _ = blk["edge"]["layers"]
    g, beta = blk["edge"]["ln"]
    aW, ab = blk["att"]
    W1e = W1[256:].astype(BF16)
    aWr = aW.reshape(1, 128)
    ab2 = ab.reshape(1, 1)

    def body(p_ref, e_ref, w1e, b1_, w2, b2_, w3, b3_,
             g_, be_, aw, ab_, w_ref, x_ref, *res):
        e = e_ref[...]
        h = p_ref[...] + jnp.dot(e.astype(BF16), w1e[...],
                                 preferred_element_type=F32)
        h = jnp.maximum(h, 0.0)
        h = jnp.maximum(h @ w2[...] + b2_[...], 0.0)
        ne = _ln_rows(h @ w3[...] + b3_[...], g_[...], be_[...])
        lg = jnp.sum(ne * aw[...], axis=-1, keepdims=True) + ab_[...]
        lg = jnp.where(lg >= 0, lg, 0.2 * lg)
        ex = jnp.exp(lg)
        w_ref[...] = ne * ex
        x_ref[...] = ex
        if want_resid:
            res[0][...] = ne + e

    out_shape = [jax.ShapeDtypeStruct((E_PAD, 128), F32),
                 jax.ShapeDtypeStruct((E_PAD, 1), F32)]
    out_specs = [pl.BlockSpec((EB, 128), lambda i: (i, 0)),
                 pl.BlockSpec((EB, 1), lambda i: (i, 0))]
    if want_resid:
        out_shape.append(jax.ShapeDtypeStruct((E_PAD, 128), F32))
        out_specs.append(pl.BlockSpec((EB, 128), lambda i: (i, 0)))

    return pl.pallas_call(
        body,
        grid=(E_PAD // EB,),
        in_specs=[
            pl.BlockSpec((EB, 128), lambda i: (i, 0)),
            pl.BlockSpec((EB, 128), lambda i: (i, 0)),
            _full((128, 128)), _full((128,)),
            _full((128, 128)), _full((128,)),
            _full((128, 128)), _full((128,)),
            _full((128,)), _full((128,)), _full((1, 128)), _full((1, 1)),
        ],
        out_specs=out_specs,
        out_shape=out_shape,
    )(pre, elat, W1e, blk["edge"]["layers"][0][1], W2, b2, W3, b3, g, beta,
      aWr, ab2)


def _node_tc(nlat, a0, a1, s0, s1, p):
    (W1, b1), (W2, b2), (W3, b3) = p["layers"]
    g, beta = p["ln"]
    W1n, W1a = W1[:128], W1[128:]

    def body(n_ref, a0_, a1_, s0_, s1_, w1n, w1a, b1_, w2, b2_, w3, b3_,
             g_, be_, o_ref):
        n = n_ref[...]
        agg = (a0_[...] + a1_[...]) / (s0_[...] + s1_[...] + 1e-16)
        h = jnp.maximum(n @ w1n[...] + agg @ w1a[...] + b1_[...], 0.0)
        h = jnp.maximum(h @ w2[...] + b2_[...], 0.0)
        y = _ln_rows(h @ w3[...] + b3_[...], g_[...], be_[...])
        o_ref[...] = y + n

    return pl.pallas_call(
        body,
        grid=(N_NODES // NB,),
        in_specs=[
            pl.BlockSpec((NB, 128), lambda i: (i, 0)),
            pl.BlockSpec((NB, 128), lambda i: (i, 0)),
            pl.BlockSpec((NB, 128), lambda i: (i, 0)),
            pl.BlockSpec((NB, 1), lambda i: (i, 0)),
            pl.BlockSpec((NB, 1), lambda i: (i, 0)),
            _full((128, 128)), _full((128, 128)), _full((128,)),
            _full((128, 128)), _full((128,)), _full((128, 128)),
            _full((128,)), _full((128,)), _full((128,)),
        ],
        out_specs=pl.BlockSpec((NB, 128), lambda i: (i, 0)),
        out_shape=jax.ShapeDtypeStruct((N_NODES, 128), F32),
    )(nlat, a0, a1, s0, s1, W1n, W1a, b1, W2, b2, W3, b3, g, beta)


def _dec_tc(nlat, p):
    (W1, b1), (W2, b2), (W3, b3) = p["layers"]
    W3p = jnp.zeros((128, 128), F32).at[:, :W3.shape[1]].set(W3)
    b3p = jnp.zeros((128,), F32).at[:b3.shape[0]].set(b3)

    def body(n_ref, w1, b1_, w2, b2_, w3, b3_, o_ref):
        h = jnp.maximum(n_ref[...] @ w1[...] + b1_[...], 0.0)
        h = jnp.maximum(h @ w2[...] + b2_[...], 0.0)
        o_ref[...] = h @ w3[...] + b3_[...]

    return pl.pallas_call(
        body,
        grid=(N_NODES // NB,),
        in_specs=[
            pl.BlockSpec((NB, 128), lambda i: (i, 0)),
            _full((128, 128)), _full((128,)), _full((128, 128)),
            _full((128,)), _full((128, 128)), _full((128,)),
        ],
        out_specs=pl.BlockSpec((NB, 128), lambda i: (i, 0)),
        out_shape=jax.ShapeDtypeStruct((N_NODES, 128), F32),
    )(nlat, W1, b1, W2, b2, W3p, b3p)


def _sc_gather(ps, pr, ix_s3, ix_r3):
    mesh = plsc.VectorSubcoreMesh(core_axis_name="c", subcore_axis_name="s")

    @functools.partial(
        pl.kernel,
        out_type=jax.ShapeDtypeStruct((E_PAD, 128), F32),
        mesh=mesh,
        scratch_types=[
            pltpu.VMEM((SID_CH, CHUNK), jnp.int32),
            pltpu.VMEM((SID_CH, CHUNK), jnp.int32),
            pltpu.VMEM((CHUNK, 128), F32),
            pltpu.VMEM((CHUNK, 128), F32),
            pltpu.VMEM((CHUNK, 128), F32),
            pltpu.VMEM((CHUNK, 128), F32),
            [pltpu.SemaphoreType.DMA] * 6,
        ],
    )
    def k(ps_hbm, pr_hbm, ixs_hbm, ixr_hbm, o_hbm,
          ixs_v, ixr_v, bsA, brA, bsB, brB, sems):
        gsA, grA, gsB, grB, oA, oB = sems
        cid = lax.axis_index("c")
        sid = lax.axis_index("s")
        pltpu.sync_copy(ixs_hbm.at[sid], ixs_v)
        pltpu.sync_copy(ixr_hbm.at[sid], ixr_v)
        loc0 = jnp.where(cid == 0, 0, GK0)
        nch = jnp.where(cid == 0, GK0, SID_CH - GK0)
        base = (sid * SID_CH + loc0) * CHUNK

        def g_start(bs, br, sem_s, sem_r, c):
            pltpu.async_copy(ps_hbm.at[ixs_v.at[loc0 + c]], bs, sem_s)
            pltpu.async_copy(pr_hbm.at[ixr_v.at[loc0 + c]], br, sem_r)

        def g_wait(bs, br, sem_s, sem_r, c):
            pltpu.make_async_copy(
                ps_hbm.at[ixs_v.at[loc0 + c]], bs, sem_s).wait()
            pltpu.make_async_copy(
                pr_hbm.at[ixr_v.at[loc0 + c]], br, sem_r).wait()

        def add_rows(bs, br):
            def row(i, carry):
                for kk in range(8):
                    sl = pl.ds(kk * 16, 16)
                    bs[i, sl] = bs[i, sl] + br[i, sl]
                return carry
            lax.fori_loop(0, CHUNK, row, 0)

        def o_start(bs, sem, c):
            pltpu.async_copy(
                bs, o_hbm.at[pl.ds(base + c * CHUNK, CHUNK)], sem)

        def o_wait(bs, sem, c):
            pltpu.make_async_copy(
                bs, o_hbm.at[pl.ds(base + c * CHUNK, CHUNK)], sem).wait()

        g_start(bsA, brA, gsA, grA, 0)
        g_start(bsB, brB, gsB, grB, 1)

        def body(j, carry):
            c0 = 2 * j
            c1 = c0 + 1
            g_wait(bsA, brA, gsA, grA, c0)
            add_rows(bsA, brA)
            o_start(bsA, oA, c0)
            g_wait(bsB, brB, gsB, grB, c1)
            add_rows(bsB, brB)
            o_start(bsB, oB, c1)
            o_wait(bsA, oA, c0)
            g_start(bsA, brA, gsA, grA, c0 + 2)
            o_wait(bsB, oB, c1)
            g_start(bsB, brB, gsB, grB, c1 + 2)
            return carry

        lax.fori_loop(0, nch // 2 - 1, body, 0)
        c0 = nch - 2
        c1 = nch - 1
        g_wait(bsA, brA, gsA, grA, c0)
        add_rows(bsA, brA)
        o_start(bsA, oA, c0)
        g_wait(bsB, brB, gsB, grB, c1)
        add_rows(bsB, brB)
        o_start(bsB, oB, c1)
        o_wait(bsA, oA, c0)
        o_wait(bsB, oB, c1)

    return k(ps, pr, ix_s3, ix_r3)


def _sc_scatter(wrows, expl, ix_r3):
    mesh = plsc.VectorSubcoreMesh(core_axis_name="c", subcore_axis_name="s")

    @functools.partial(
        pl.kernel,
        out_type=(jax.ShapeDtypeStruct((N_SP, 128), F32),
                  jax.ShapeDtypeStruct((N_SP, 128), F32),
                  jax.ShapeDtypeStruct((N_SP,), F32),
                  jax.ShapeDtypeStruct((N_SP,), F32)),
        mesh=mesh,
        scratch_types=[
            pltpu.VMEM_SHARED((N_SP, 128), F32),
            pltpu.VMEM_SHARED((N_SP,), F32),
            pltpu.VMEM((CH_PER_W, CHUNK), jnp.int32),
            pltpu.VMEM((E_PER_W,), F32),
            pltpu.VMEM((CHUNK, 128), F32),
            pltpu.VMEM((CHUNK, 128), F32),
            [pltpu.SemaphoreType.DMA] * 4,
        ],
    )
    def k(w_hbm, ex_hbm, ix_hbm, a0_hbm, a1_hbm, s0_hbm, s1_hbm,
          spa, sps, ix_v, ex_v, bufA, bufB, sems):
        lA, lB, sA, sB = sems
        cid = lax.axis_index("c")
        sid = lax.axis_index("s")

        def zrow(i, carry):
            for j in range(8):
                bufA[i, pl.ds(j * 16, 16)] = jnp.zeros((16,), F32)
            return carry

        lax.fori_loop(0, CHUNK, zrow, 0)
        z0 = sid * ZROWS
        for kk in range(ZROWS // CHUNK):
            pltpu.sync_copy(bufA, spa.at[pl.ds(z0 + kk * CHUNK, CHUNK)])
            pltpu.sync_copy(bufA.at[0], sps.at[pl.ds(z0 + kk * CHUNK, CHUNK)])
        plsc.subcore_barrier()

        pltpu.sync_copy(ix_hbm.at[sid, pl.ds(cid * CH_PER_W, CH_PER_W)], ix_v)
        base = (sid * SID_CH + cid * CH_PER_W) * CHUNK
        pltpu.sync_copy(ex_hbm.at[pl.ds(base, E_PER_W)], ex_v)

        def l_start(buf, sem, c):
            pltpu.async_copy(
                w_hbm.at[pl.ds(base + c * CHUNK, CHUNK)], buf, sem)

        def l_wait(buf, sem, c):
            pltpu.make_async_copy(
                w_hbm.at[pl.ds(base + c * CHUNK, CHUNK)], buf, sem).wait()

        def s_start(buf, sem, c):
            pltpu.async_copy(buf, spa.at[ix_v.at[c]], sem, add=True)
            pltpu.async_copy(
                ex_v.at[pl.ds(c * CHUNK, CHUNK)], sps.at[ix_v.at[c]],
                sem, add=True)

        def s_wait(buf, sem, c):
            pltpu.make_async_copy(buf, spa.at[ix_v.at[c]], sem).wait()
            pltpu.make_async_copy(
                ex_v.at[pl.ds(c * CHUNK, CHUNK)], sps.at[ix_v.at[c]],
                sem).wait()

        l_start(bufA, lA, 0)
        l_start(bufB, lB, 1)

        def step(j, carry):
            c0 = 2 * j
            c1 = c0 + 1
            l_wait(bufA, lA, c0)
            s_start(bufA, sA, c0)
            l_wait(bufB, lB, c1)
            s_start(bufB, sB, c1)
            s_wait(bufA, sA, c0)
            l_start(bufA, lA, c0 + 2)
            s_wait(bufB, sB, c1)
            l_start(bufB, lB, c1 + 2)
            return carry

        lax.fori_loop(0, CH_PER_W // 2 - 1, step, 0)
        c0 = CH_PER_W - 2
        c1 = CH_PER_W - 1
        l_wait(bufA, lA, c0)
        s_start(bufA, sA, c0)
        l_wait(bufB, lB, c1)
        s_start(bufB, sB, c1)
        s_wait(bufA, sA, c0)
        s_wait(bufB, sB, c1)
        plsc.subcore_barrier()

        @pl.when(cid == 0)
        def _():
            pltpu.sync_copy(spa.at[pl.ds(z0, ZROWS)], a0_hbm.at[pl.ds(z0, ZROWS)])
            pltpu.sync_copy(sps.at[pl.ds(z0, ZROWS)], s0_hbm.at[pl.ds(z0, ZROWS)])

        @pl.when(cid == 1)
        def _():
            pltpu.sync_copy(spa.at[pl.ds(z0, ZROWS)], a1_hbm.at[pl.ds(z0, ZROWS)])
            pltpu.sync_copy(sps.at[pl.ds(z0, ZROWS)], s1_hbm.at[pl.ds(z0, ZROWS)])

    return k(wrows, expl, ix_r3)


def kernel(node_features, edge_features, senders, receivers, image_feature,
           params):
    pad = E_PAD - N_EDGES
    s_pad = jnp.concatenate([senders, jnp.zeros((pad,), jnp.int32)])
    r_pad = jnp.concatenate([receivers, jnp.full((pad,), N_NODES, jnp.int32)])
    ix_s3 = s_pad.reshape(16, SID_CH, CHUNK)
    ix_r3 = r_pad.reshape(16, SID_CH, CHUNK)

    node_lat = _enc_node(node_features, image_feature, params["enc_node"])
    edge_lat = _enc_edge(edge_features, params["enc_edge"])

    for i, blk in enumerate(params["blocks"]):
        ps, pr = _proj(node_lat, blk)
        pre = _sc_gather(ps, pr, ix_s3, ix_r3)
        outs = _edge_tc(pre, edge_lat, blk, want_resid=(i == 0))
        wrows, expl = outs[0], outs[1]
        if i == 0:
            edge_lat = outs[2]
        a0, a1, ss0, ss1 = _sc_scatter(wrows, expl.reshape(E_PAD), ix_r3)
        node_lat = _node_tc(node_lat, a0, a1,
                            ss0.reshape(N_SP, 1), ss1.reshape(N_SP, 1),
                            blk["node"])

    dec = _dec_tc(node_lat, params["dec"])
    return dec[:, :3].reshape(1, N_NODES, 3)


# speedup vs baseline: 4.6061x; 1.0001x over previous
import functools

import jax
import jax.numpy as jnp
from jax import lax
from jax.experimental import pallas as pl
from jax.experimental.pallas import tpu as pltpu
from jax.experimental.pallas import tpu_sc as plsc

N_NODES = 10000
N_SP = 10240
N_EDGES = 160000
CHUNK = 128
CH_PER_W = 40
E_PER_W = CHUNK * CH_PER_W
E_PAD = 32 * E_PER_W
EB = 1280
NB = 2000
ZROWS = N_SP // 16
SID_CH = 80
GK0 = 60
F32 = jnp.float32
BF16 = jnp.bfloat16


def _ln_rows(x, g, beta):
    mu = jnp.mean(x, axis=-1, keepdims=True)
    var = jnp.mean((x - mu) ** 2, axis=-1, keepdims=True)
    return (x - mu) / jnp.sqrt(var + 1e-5) * g + beta


def _full(shape):
    nd = len(shape)
    return pl.BlockSpec(shape, lambda i: (0,) * nd)


def _enc_node(nf, img, p):
    (W1, b1), (W2, b2) = p["layers"]
    g, beta = p["ln"]
    W1a, W1b = W1[:128], W1[128:]

    def body(nf_ref, img_ref, w1a, w1b, b1_, w2, b2_, g_, be_, o_ref):
        h = nf_ref[...] @ w1a[...] + img_ref[...] @ w1b[...] + b1_[...]
        h = jnp.maximum(h, 0.0)
        y = h @ w2[...] + b2_[...]
        o_ref[...] = _ln_rows(y, g_[...], be_[...])

    return pl.pallas_call(
        body,
        grid=(N_NODES // NB,),
        in_specs=[
            pl.BlockSpec((NB, 128), lambda i: (i, 0)),
            _full((1, 512)), _full((128, 256)), _full((512, 256)),
            _full((256,)), _full((256, 128)), _full((128,)),
            _full((128,)), _full((128,)),
        ],
        out_specs=pl.BlockSpec((NB, 128), lambda i: (i, 0)),
        out_shape=jax.ShapeDtypeStruct((N_NODES, 128), F32),
    )(nf, img, W1a, W1b, b1, W2, b2, g, beta)


def _enc_edge(ef, p):
    (W1, b1), (W2, b2) = p["layers"]
    g, beta = p["ln"]

    def body(ef_ref, w1, b1_, w2, b2_, g_, be_, o_ref):
        h = ef_ref[...] @ w1[...] + b1_[...]
        h = jnp.maximum(h, 0.0)
        y = h @ w2[...] + b2_[...]
        o_ref[...] = _ln_rows(y, g_[...], be_[...])

    last = N_EDGES // EB - 1
    return pl.pallas_call(
        body,
        grid=(E_PAD // EB,),
        in_specs=[
            pl.BlockSpec((EB, 16), lambda i: (jnp.minimum(i, last), 0)),
            _full((16, 256)), _full((256,)), _full((256, 128)),
            _full((128,)), _full((128,)), _full((128,)),
        ],
        out_specs=pl.BlockSpec((EB, 128), lambda i: (i, 0)),
        out_shape=jax.ShapeDtypeStruct((E_PAD, 128), F32),
    )(ef, W1, b1, W2, b2, g, beta)


def _proj(nlat, blk):
    (W1, b1), _, _ = blk["edge"]["layers"]
    W1s, W1r = W1[:128], W1[128:256]

    def body(n_ref, w1s, w1r, b1_, ps_ref, pr_ref):
        n = n_ref[...]
        ps_ref[...] = n @ w1s[...] + b1_[...]
        pr_ref[...] = n @ w1r[...]

    return pl.pallas_call(
        body,
        grid=(N_NODES // NB,),
        in_specs=[
            pl.BlockSpec((NB, 128), lambda i: (i, 0)),
            _full((128, 128)), _full((128, 128)), _full((128,)),
        ],
        out_specs=[pl.BlockSpec((NB, 128), lambda i: (i, 0)),
                   pl.BlockSpec((NB, 128), lambda i: (i, 0))],
        out_shape=[jax.ShapeDtypeStruct((N_NODES, 128), F32),
                   jax.ShapeDtypeStruct((N_NODES, 128), F32)],
    )(nlat, W1s, W1r, b1)


def _edge_tc(pre, elat, blk, want_resid):
    _, (W2, b2), (W3, b3) = blk["edge"]["layers"]
    (W1, _), _, _ = blk["edge"]["layers"]
    g, beta = blk["edge"]["ln"]
    aW, ab = blk["att"]
    W1e = W1[256:].astype(BF16)
    W2 = W2.astype(BF16)
    aWr = aW.reshape(1, 128)
    ab2 = ab.reshape(1, 1)

    def body(p_ref, e_ref, w1e, b1_, w2, b2_, w3, b3_,
             g_, be_, aw, ab_, w_ref, x_ref, *res):
        e = e_ref[...]
        h = p_ref[...] + jnp.dot(e.astype(BF16), w1e[...],
                                 preferred_element_type=F32)
        h = jnp.maximum(h, 0.0)
        h = jnp.maximum(
            jnp.dot(h.astype(BF16), w2[...], preferred_element_type=F32)
            + b2_[...], 0.0)
        ne = _ln_rows(h @ w3[...] + b3_[...], g_[...], be_[...])
        lg = jnp.sum(ne * aw[...], axis=-1, keepdims=True) + ab_[...]
        lg = jnp.where(lg >= 0, lg, 0.2 * lg)
        ex = jnp.exp(lg)
        w_ref[...] = ne * ex
        x_ref[...] = ex
        if want_resid:
            res[0][...] = ne + e

    out_shape = [jax.ShapeDtypeStruct((E_PAD, 128), F32),
                 jax.ShapeDtypeStruct((E_PAD, 1), F32)]
    out_specs = [pl.BlockSpec((EB, 128), lambda i: (i, 0)),
                 pl.BlockSpec((EB, 1), lambda i: (i, 0))]
    if want_resid:
        out_shape.append(jax.ShapeDtypeStruct((E_PAD, 128), F32))
        out_specs.append(pl.BlockSpec((EB, 128), lambda i: (i, 0)))

    return pl.pallas_call(
        body,
        grid=(E_PAD // EB,),
        in_specs=[
            pl.BlockSpec((EB, 128), lambda i: (i, 0)),
            pl.BlockSpec((EB, 128), lambda i: (i, 0)),
            _full((128, 128)), _full((128,)),
            _full((128, 128)), _full((128,)),
            _full((128, 128)), _full((128,)),
            _full((128,)), _full((128,)), _full((1, 128)), _full((1, 1)),
        ],
        out_specs=out_specs,
        out_shape=out_shape,
    )(pre, elat, W1e, blk["edge"]["layers"][0][1], W2, b2, W3, b3, g, beta,
      aWr, ab2)


def _node_tc(nlat, a0, a1, s0, s1, p):
    (W1, b1), (W2, b2), (W3, b3) = p["layers"]
    g, beta = p["ln"]
    W1n, W1a = W1[:128], W1[128:]

    def body(n_ref, a0_, a1_, s0_, s1_, w1n, w1a, b1_, w2, b2_, w3, b3_,
             g_, be_, o_ref):
        n = n_ref[...]
        agg = (a0_[...] + a1_[...]) / (s0_[...] + s1_[...] + 1e-16)
        h = jnp.maximum(n @ w1n[...] + agg @ w1a[...] + b1_[...], 0.0)
        h = jnp.maximum(h @ w2[...] + b2_[...], 0.0)
        y = _ln_rows(h @ w3[...] + b3_[...], g_[...], be_[...])
        o_ref[...] = y + n

    return pl.pallas_call(
        body,
        grid=(N_NODES // NB,),
        in_specs=[
            pl.BlockSpec((NB, 128), lambda i: (i, 0)),
            pl.BlockSpec((NB, 128), lambda i: (i, 0)),
            pl.BlockSpec((NB, 128), lambda i: (i, 0)),
            pl.BlockSpec((NB, 1), lambda i: (i, 0)),
            pl.BlockSpec((NB, 1), lambda i: (i, 0)),
            _full((128, 128)), _full((128, 128)), _full((128,)),
            _full((128, 128)), _full((128,)), _full((128, 128)),
            _full((128,)), _full((128,)), _full((128,)),
        ],
        out_specs=pl.BlockSpec((NB, 128), lambda i: (i, 0)),
        out_shape=jax.ShapeDtypeStruct((N_NODES, 128), F32),
    )(nlat, a0, a1, s0, s1, W1n, W1a, b1, W2, b2, W3, b3, g, beta)


def _dec_tc(nlat, p):
    (W1, b1), (W2, b2), (W3, b3) = p["layers"]
    W3p = jnp.zeros((128, 128), F32).at[:, :W3.shape[1]].set(W3)
    b3p = jnp.zeros((128,), F32).at[:b3.shape[0]].set(b3)

    def body(n_ref, w1, b1_, w2, b2_, w3, b3_, o_ref):
        h = jnp.maximum(n_ref[...] @ w1[...] + b1_[...], 0.0)
        h = jnp.maximum(h @ w2[...] + b2_[...], 0.0)
        o_ref[...] = h @ w3[...] + b3_[...]

    return pl.pallas_call(
        body,
        grid=(N_NODES // NB,),
        in_specs=[
            pl.BlockSpec((NB, 128), lambda i: (i, 0)),
            _full((128, 128)), _full((128,)), _full((128, 128)),
            _full((128,)), _full((128, 128)), _full((128,)),
        ],
        out_specs=pl.BlockSpec((NB, 128), lambda i: (i, 0)),
        out_shape=jax.ShapeDtypeStruct((N_NODES, 128), F32),
    )(nlat, W1, b1, W2, b2, W3p, b3p)


def _sc_gather(ps, pr, ix_s3, ix_r3):
    mesh = plsc.VectorSubcoreMesh(core_axis_name="c", subcore_axis_name="s")

    @functools.partial(
        pl.kernel,
        out_type=jax.ShapeDtypeStruct((E_PAD, 128), F32),
        mesh=mesh,
        scratch_types=[
            pltpu.VMEM((SID_CH, CHUNK), jnp.int32),
            pltpu.VMEM((SID_CH, CHUNK), jnp.int32),
            pltpu.VMEM((CHUNK, 128), F32),
            pltpu.VMEM((CHUNK, 128), F32),
            pltpu.VMEM((CHUNK, 128), F32),
            pltpu.VMEM((CHUNK, 128), F32),
            [pltpu.SemaphoreType.DMA] * 6,
        ],
    )
    def k(ps_hbm, pr_hbm, ixs_hbm, ixr_hbm, o_hbm,
          ixs_v, ixr_v, bsA, brA, bsB, brB, sems):
        gsA, grA, gsB, grB, oA, oB = sems
        cid = lax.axis_index("c")
        sid = lax.axis_index("s")
        pltpu.sync_copy(ixs_hbm.at[sid], ixs_v)
        pltpu.sync_copy(ixr_hbm.at[sid], ixr_v)
        loc0 = jnp.where(cid == 0, 0, GK0)
        nch = jnp.where(cid == 0, GK0, SID_CH - GK0)
        base = (sid * SID_CH + loc0) * CHUNK

        def g_start(bs, br, sem_s, sem_r, c):
            pltpu.async_copy(ps_hbm.at[ixs_v.at[loc0 + c]], bs, sem_s)
            pltpu.async_copy(pr_hbm.at[ixr_v.at[loc0 + c]], br, sem_r)

        def g_wait(bs, br, sem_s, sem_r, c):
            pltpu.make_async_copy(
                ps_hbm.at[ixs_v.at[loc0 + c]], bs, sem_s).wait()
            pltpu.make_async_copy(
                pr_hbm.at[ixr_v.at[loc0 + c]], br, sem_r).wait()

        def add_rows(bs, br):
            def row(i, carry):
                for kk in range(8):
                    sl = pl.ds(kk * 16, 16)
                    bs[i, sl] = bs[i, sl] + br[i, sl]
                return carry
            lax.fori_loop(0, CHUNK, row, 0)

        def o_start(bs, sem, c):
            pltpu.async_copy(
                bs, o_hbm.at[pl.ds(base + c * CHUNK, CHUNK)], sem)

        def o_wait(bs, sem, c):
            pltpu.make_async_copy(
                bs, o_hbm.at[pl.ds(base + c * CHUNK, CHUNK)], sem).wait()

        g_start(bsA, brA, gsA, grA, 0)
        g_start(bsB, brB, gsB, grB, 1)

        def body(j, carry):
            c0 = 2 * j
            c1 = c0 + 1
            g_wait(bsA, brA, gsA, grA, c0)
            add_rows(bsA, brA)
            o_start(bsA, oA, c0)
            g_wait(bsB, brB, gsB, grB, c1)
            add_rows(bsB, brB)
            o_start(bsB, oB, c1)
            o_wait(bsA, oA, c0)
            g_start(bsA, brA, gsA, grA, c0 + 2)
            o_wait(bsB, oB, c1)
            g_start(bsB, brB, gsB, grB, c1 + 2)
            return carry

        lax.fori_loop(0, nch // 2 - 1, body, 0)
        c0 = nch - 2
        c1 = nch - 1
        g_wait(bsA, brA, gsA, grA, c0)
        add_rows(bsA, brA)
        o_start(bsA, oA, c0)
        g_wait(bsB, brB, gsB, grB, c1)
        add_rows(bsB, brB)
        o_start(bsB, oB, c1)
        o_wait(bsA, oA, c0)
        o_wait(bsB, oB, c1)

    return k(ps, pr, ix_s3, ix_r3)


def _sc_scatter(wrows, expl, ix_r3):
    mesh = plsc.VectorSubcoreMesh(core_axis_name="c", subcore_axis_name="s")

    @functools.partial(
        pl.kernel,
        out_type=(jax.ShapeDtypeStruct((N_SP, 128), F32),
                  jax.ShapeDtypeStruct((N_SP, 128), F32),
                  jax.ShapeDtypeStruct((N_SP,), F32),
                  jax.ShapeDtypeStruct((N_SP,), F32)),
        mesh=mesh,
        scratch_types=[
            pltpu.VMEM_SHARED((N_SP, 128), F32),
            pltpu.VMEM_SHARED((N_SP,), F32),
            pltpu.VMEM((CH_PER_W, CHUNK), jnp.int32),
            pltpu.VMEM((E_PER_W,), F32),
            pltpu.VMEM((CHUNK, 128), F32),
            pltpu.VMEM((CHUNK, 128), F32),
            [pltpu.SemaphoreType.DMA] * 4,
        ],
    )
    def k(w_hbm, ex_hbm, ix_hbm, a0_hbm, a1_hbm, s0_hbm, s1_hbm,
          spa, sps, ix_v, ex_v, bufA, bufB, sems):
        lA, lB, sA, sB = sems
        cid = lax.axis_index("c")
        sid = lax.axis_index("s")

        def zrow(i, carry):
            for j in range(8):
                bufA[i, pl.ds(j * 16, 16)] = jnp.zeros((16,), F32)
            return carry

        lax.fori_loop(0, CHUNK, zrow, 0)
        z0 = sid * ZROWS
        for kk in range(ZROWS // CHUNK):
            pltpu.sync_copy(bufA, spa.at[pl.ds(z0 + kk * CHUNK, CHUNK)])
            pltpu.sync_copy(bufA.at[0], sps.at[pl.ds(z0 + kk * CHUNK, CHUNK)])
        plsc.subcore_barrier()

        pltpu.sync_copy(ix_hbm.at[sid, pl.ds(cid * CH_PER_W, CH_PER_W)], ix_v)
        base = (sid * SID_CH + cid * CH_PER_W) * CHUNK
        pltpu.sync_copy(ex_hbm.at[pl.ds(base, E_PER_W)], ex_v)

        def l_start(buf, sem, c):
            pltpu.async_copy(
                w_hbm.at[pl.ds(base + c * CHUNK, CHUNK)], buf, sem)

        def l_wait(buf, sem, c):
            pltpu.make_async_copy(
                w_hbm.at[pl.ds(base + c * CHUNK, CHUNK)], buf, sem).wait()

        def s_start(buf, sem, c):
            pltpu.async_copy(buf, spa.at[ix_v.at[c]], sem, add=True)
            pltpu.async_copy(
                ex_v.at[pl.ds(c * CHUNK, CHUNK)], sps.at[ix_v.at[c]],
                sem, add=True)

        def s_wait(buf, sem, c):
            pltpu.make_async_copy(buf, spa.at[ix_v.at[c]], sem).wait()
            pltpu.make_async_copy(
                ex_v.at[pl.ds(c * CHUNK, CHUNK)], sps.at[ix_v.at[c]],
                sem).wait()

        l_start(bufA, lA, 0)
        l_start(bufB, lB, 1)

        def step(j, carry):
            c0 = 2 * j
            c1 = c0 + 1
            l_wait(bufA, lA, c0)
            s_start(bufA, sA, c0)
            l_wait(bufB, lB, c1)
            s_start(bufB, sB, c1)
            s_wait(bufA, sA, c0)
            l_start(bufA, lA, c0 + 2)
            s_wait(bufB, sB, c1)
            l_start(bufB, lB, c1 + 2)
            return carry

        lax.fori_loop(0, CH_PER_W // 2 - 1, step, 0)
        c0 = CH_PER_W - 2
        c1 = CH_PER_W - 1
        l_wait(bufA, lA, c0)
        s_start(bufA, sA, c0)
        l_wait(bufB, lB, c1)
        s_start(bufB, sB, c1)
        s_wait(bufA, sA, c0)
        s_wait(bufB, sB, c1)
        plsc.subcore_barrier()

        @pl.when(cid == 0)
        def _():
            pltpu.sync_copy(spa.at[pl.ds(z0, ZROWS)], a0_hbm.at[pl.ds(z0, ZROWS)])
            pltpu.sync_copy(sps.at[pl.ds(z0, ZROWS)], s0_hbm.at[pl.ds(z0, ZROWS)])

        @pl.when(cid == 1)
        def _():
            pltpu.sync_copy(spa.at[pl.ds(z0, ZROWS)], a1_hbm.at[pl.ds(z0, ZROWS)])
            pltpu.sync_copy(sps.at[pl.ds(z0, ZROWS)], s1_hbm.at[pl.ds(z0, ZROWS)])

    return k(wrows, expl, ix_r3)


def kernel(node_features, edge_features, senders, receivers, image_feature,
           params):
    pad = E_PAD - N_EDGES
    s_pad = jnp.concatenate([senders, jnp.zeros((pad,), jnp.int32)])
    r_pad = jnp.concatenate([receivers, jnp.full((pad,), N_NODES, jnp.int32)])
    ix_s3 = s_pad.reshape(16, SID_CH, CHUNK)
    ix_r3 = r_pad.reshape(16, SID_CH, CHUNK)

    node_lat = _enc_node(node_features, image_feature, params["enc_node"])
    edge_lat = _enc_edge(edge_features, params["enc_edge"])

    for i, blk in enumerate(params["blocks"]):
        ps, pr = _proj(node_lat, blk)
        pre = _sc_gather(ps, pr, ix_s3, ix_r3)
        outs = _edge_tc(pre, edge_lat, blk, want_resid=(i == 0))
        wrows, expl = outs[0], outs[1]
        if i == 0:
            edge_lat = outs[2]
        a0, a1, ss0, ss1 = _sc_scatter(wrows, expl.reshape(E_PAD), ix_r3)
        node_lat = _node_tc(node_lat, a0, a1,
                            ss0.reshape(N_SP, 1), ss1.reshape(N_SP, 1),
                            blk["node"])

    dec = _dec_tc(node_lat, params["dec"])
    return dec[:, :3].reshape(1, N_NODES, 3)


# speedup vs baseline: 4.6541x; 1.0104x over previous
import functools

import jax
import jax.numpy as jnp
from jax import lax
from jax.experimental import pallas as pl
from jax.experimental.pallas import tpu as pltpu
from jax.experimental.pallas import tpu_sc as plsc

N_NODES = 10000
N_SP = 10240
N_EDGES = 160000
CHUNK = 128
CH_PER_W = 40
E_PER_W = CHUNK * CH_PER_W
E_PAD = 32 * E_PER_W
EB = 1280
NB = 2000
ZROWS = N_SP // 16
SID_CH = 80
E_H = E_PAD // 2
SID_CH_H = 40
GK0 = 30
F32 = jnp.float32
BF16 = jnp.bfloat16


def _ln_rows(x, g, beta):
    mu = jnp.mean(x, axis=-1, keepdims=True)
    var = jnp.mean((x - mu) ** 2, axis=-1, keepdims=True)
    return (x - mu) / jnp.sqrt(var + 1e-5) * g + beta


def _full(shape):
    nd = len(shape)
    return pl.BlockSpec(shape, lambda i: (0,) * nd)


def _enc_node(nf, img, p):
    (W1, b1), (W2, b2) = p["layers"]
    g, beta = p["ln"]
    W1a, W1b = W1[:128], W1[128:]

    def body(nf_ref, img_ref, w1a, w1b, b1_, w2, b2_, g_, be_, o_ref):
        h = nf_ref[...] @ w1a[...] + img_ref[...] @ w1b[...] + b1_[...]
        h = jnp.maximum(h, 0.0)
        y = h @ w2[...] + b2_[...]
        o_ref[...] = _ln_rows(y, g_[...], be_[...])

    return pl.pallas_call(
        body,
        grid=(N_NODES // NB,),
        in_specs=[
            pl.BlockSpec((NB, 128), lambda i: (i, 0)),
            _full((1, 512)), _full((128, 256)), _full((512, 256)),
            _full((256,)), _full((256, 128)), _full((128,)),
            _full((128,)), _full((128,)),
        ],
        out_specs=pl.BlockSpec((NB, 128), lambda i: (i, 0)),
        out_shape=jax.ShapeDtypeStruct((N_NODES, 128), F32),
    )(nf, img, W1a, W1b, b1, W2, b2, g, beta)


def _enc_edge(ef, p):
    (W1, b1), (W2, b2) = p["layers"]
    g, beta = p["ln"]

    def body(ef_ref, w1, b1_, w2, b2_, g_, be_, o_ref):
        h = ef_ref[...] @ w1[...] + b1_[...]
        h = jnp.maximum(h, 0.0)
        y = h @ w2[...] + b2_[...]
        o_ref[...] = _ln_rows(y, g_[...], be_[...])

    last = N_EDGES // EB - 1
    return pl.pallas_call(
        body,
        grid=(E_PAD // EB,),
        in_specs=[
            pl.BlockSpec((EB, 16), lambda i: (jnp.minimum(i, last), 0)),
            _full((16, 256)), _full((256,)), _full((256, 128)),
            _full((128,)), _full((128,)), _full((128,)),
        ],
        out_specs=pl.BlockSpec((EB, 128), lambda i: (i, 0)),
        out_shape=jax.ShapeDtypeStruct((E_PAD, 128), F32),
    )(ef, W1, b1, W2, b2, g, beta)


def _proj(nlat, blk):
    (W1, b1), _, _ = blk["edge"]["layers"]
    W1s, W1r = W1[:128], W1[128:256]

    def body(n_ref, w1s, w1r, b1_, ps_ref, pr_ref):
        n = n_ref[...]
        ps_ref[...] = n @ w1s[...] + b1_[...]
        pr_ref[...] = n @ w1r[...]

    return pl.pallas_call(
        body,
        grid=(N_NODES // NB,),
        in_specs=[
            pl.BlockSpec((NB, 128), lambda i: (i, 0)),
            _full((128, 128)), _full((128, 128)), _full((128,)),
        ],
        out_specs=[pl.BlockSpec((NB, 128), lambda i: (i, 0)),
                   pl.BlockSpec((NB, 128), lambda i: (i, 0))],
        out_shape=[jax.ShapeDtypeStruct((N_NODES, 128), F32),
                   jax.ShapeDtypeStruct((N_NODES, 128), F32)],
    )(nlat, W1s, W1r, b1)


def _edge_tc(pre, elat, blk, want_resid, e_off):
    _, (W2, b2), (W3, b3) = blk["edge"]["layers"]
    (W1, _), _, _ = blk["edge"]["layers"]
    g, beta = blk["edge"]["ln"]
    aW, ab = blk["att"]
    W1e = W1[256:].astype(BF16)
    aWr = aW.reshape(1, 128)
    ab2 = ab.reshape(1, 1)

    def body(p_ref, e_ref, w1e, b1_, w2, b2_, w3, b3_,
             g_, be_, aw, ab_, w_ref, x_ref, *res):
        e = e_ref[...]
        h = p_ref[...] + jnp.dot(e.astype(BF16), w1e[...],
                                 preferred_element_type=F32)
        h = jnp.maximum(h, 0.0)
        h = jnp.maximum(h @ w2[...] + b2_[...], 0.0)
        ne = _ln_rows(h @ w3[...] + b3_[...], g_[...], be_[...])
        lg = jnp.sum(ne * aw[...], axis=-1, keepdims=True) + ab_[...]
        lg = jnp.where(lg >= 0, lg, 0.2 * lg)
        ex = jnp.exp(lg)
        w_ref[...] = ne * ex
        x_ref[...] = ex
        if want_resid:
            res[0][...] = ne + e

    out_shape = [jax.ShapeDtypeStruct((E_H, 128), F32),
                 jax.ShapeDtypeStruct((E_H, 1), F32)]
    out_specs = [pl.BlockSpec((EB, 128), lambda i: (i, 0)),
                 pl.BlockSpec((EB, 1), lambda i: (i, 0))]
    if want_resid:
        out_shape.append(jax.ShapeDtypeStruct((E_H, 128), F32))
        out_specs.append(pl.BlockSpec((EB, 128), lambda i: (i, 0)))

    return pl.pallas_call(
        body,
        grid=(E_H // EB,),
        in_specs=[
            pl.BlockSpec((EB, 128), lambda i: (i, 0)),
            pl.BlockSpec((EB, 128), lambda i: (i + e_off, 0)),
            _full((128, 128)), _full((128,)),
            _full((128, 128)), _full((128,)),
            _full((128, 128)), _full((128,)),
            _full((128,)), _full((128,)), _full((1, 128)), _full((1, 1)),
        ],
        out_specs=out_specs,
        out_shape=out_shape,
    )(pre, elat, W1e, blk["edge"]["layers"][0][1], W2, b2, W3, b3, g, beta,
      aWr, ab2)


def _node_tc(nlat, a0, a1, s0, s1, p):
    (W1, b1), (W2, b2), (W3, b3) = p["layers"]
    g, beta = p["ln"]
    W1n, W1a = W1[:128], W1[128:]

    def body(n_ref, a0_, a1_, s0_, s1_, w1n, w1a, b1_, w2, b2_, w3, b3_,
             g_, be_, o_ref):
        n = n_ref[...]
        agg = (a0_[...] + a1_[...]) / (s0_[...] + s1_[...] + 1e-16)
        h = jnp.maximum(n @ w1n[...] + agg @ w1a[...] + b1_[...], 0.0)
        h = jnp.maximum(h @ w2[...] + b2_[...], 0.0)
        y = _ln_rows(h @ w3[...] + b3_[...], g_[...], be_[...])
        o_ref[...] = y + n

    return pl.pallas_call(
        body,
        grid=(N_NODES // NB,),
        in_specs=[
            pl.BlockSpec((NB, 128), lambda i: (i, 0)),
            pl.BlockSpec((NB, 128), lambda i: (i, 0)),
            pl.BlockSpec((NB, 128), lambda i: (i, 0)),
            pl.BlockSpec((NB, 1), lambda i: (i, 0)),
            pl.BlockSpec((NB, 1), lambda i: (i, 0)),
            _full((128, 128)), _full((128, 128)), _full((128,)),
            _full((128, 128)), _full((128,)), _full((128, 128)),
            _full((128,)), _full((128,)), _full((128,)),
        ],
        out_specs=pl.BlockSpec((NB, 128), lambda i: (i, 0)),
        out_shape=jax.ShapeDtypeStruct((N_NODES, 128), F32),
    )(nlat, a0, a1, s0, s1, W1n, W1a, b1, W2, b2, W3, b3, g, beta)


def _dec_tc(nlat, p):
    (W1, b1), (W2, b2), (W3, b3) = p["layers"]
    W3p = jnp.zeros((128, 128), F32).at[:, :W3.shape[1]].set(W3)
    b3p = jnp.zeros((128,), F32).at[:b3.shape[0]].set(b3)

    def body(n_ref, w1, b1_, w2, b2_, w3, b3_, o_ref):
        h = jnp.maximum(n_ref[...] @ w1[...] + b1_[...], 0.0)
        h = jnp.maximum(h @ w2[...] + b2_[...], 0.0)
        o_ref[...] = h @ w3[...] + b3_[...]

    return pl.pallas_call(
        body,
        grid=(N_NODES // NB,),
        in_specs=[
            pl.BlockSpec((NB, 128), lambda i: (i, 0)),
            _full((128, 128)), _full((128,)), _full((128, 128)),
            _full((128,)), _full((128, 128)), _full((128,)),
        ],
        out_specs=pl.BlockSpec((NB, 128), lambda i: (i, 0)),
        out_shape=jax.ShapeDtypeStruct((N_NODES, 128), F32),
    )(nlat, W1, b1, W2, b2, W3p, b3p)


def _sc_gather(ps, pr, ix_s3, ix_r3):
    mesh = plsc.VectorSubcoreMesh(core_axis_name="c", subcore_axis_name="s")

    @functools.partial(
        pl.kernel,
        out_type=jax.ShapeDtypeStruct((E_H, 128), F32),
        mesh=mesh,
        scratch_types=[
            pltpu.VMEM((SID_CH_H, CHUNK), jnp.int32),
            pltpu.VMEM((SID_CH_H, CHUNK), jnp.int32),
            pltpu.VMEM((CHUNK, 128), F32),
            pltpu.VMEM((CHUNK, 128), F32),
            pltpu.VMEM((CHUNK, 128), F32),
            pltpu.VMEM((CHUNK, 128), F32),
            [pltpu.SemaphoreType.DMA] * 6,
        ],
    )
    def k(ps_hbm, pr_hbm, ixs_hbm, ixr_hbm, o_hbm,
          ixs_v, ixr_v, bsA, brA, bsB, brB, sems):
        gsA, grA, gsB, grB, oA, oB = sems
        cid = lax.axis_index("c")
        sid = lax.axis_index("s")
        pltpu.sync_copy(ixs_hbm.at[sid], ixs_v)
        pltpu.sync_copy(ixr_hbm.at[sid], ixr_v)
        loc0 = jnp.where(cid == 0, 0, GK0)
        nch = jnp.where(cid == 0, GK0, SID_CH_H - GK0)
        base = (sid * SID_CH_H + loc0) * CHUNK

        def g_start(bs, br, sem_s, sem_r, c):
            pltpu.async_copy(ps_hbm.at[ixs_v.at[loc0 + c]], bs, sem_s)
            pltpu.async_copy(pr_hbm.at[ixr_v.at[loc0 + c]], br, sem_r)

        def g_wait(bs, br, sem_s, sem_r, c):
            pltpu.make_async_copy(
                ps_hbm.at[ixs_v.at[loc0 + c]], bs, sem_s).wait()
            pltpu.make_async_copy(
                pr_hbm.at[ixr_v.at[loc0 + c]], br, sem_r).wait()

        def add_rows(bs, br):
            def row(i, carry):
                for kk in range(8):
                    sl = pl.ds(kk * 16, 16)
                    bs[i, sl] = bs[i, sl] + br[i, sl]
                return carry
            lax.fori_loop(0, CHUNK, row, 0)

        def o_start(bs, sem, c):
            pltpu.async_copy(
                bs, o_hbm.at[pl.ds(base + c * CHUNK, CHUNK)], sem)

        def o_wait(bs, sem, c):
            pltpu.make_async_copy(
                bs, o_hbm.at[pl.ds(base + c * CHUNK, CHUNK)], sem).wait()

        g_start(bsA, brA, gsA, grA, 0)
        g_start(bsB, brB, gsB, grB, 1)

        def body(j, carry):
            c0 = 2 * j
            c1 = c0 + 1
            g_wait(bsA, brA, gsA, grA, c0)
            add_rows(bsA, brA)
            o_start(bsA, oA, c0)
            g_wait(bsB, brB, gsB, grB, c1)
            add_rows(bsB, brB)
            o_start(bsB, oB, c1)
            o_wait(bsA, oA, c0)
            g_start(bsA, brA, gsA, grA, c0 + 2)
            o_wait(bsB, oB, c1)
            g_start(bsB, brB, gsB, grB, c1 + 2)
            return carry

        lax.fori_loop(0, nch // 2 - 1, body, 0)
        c0 = nch - 2
        c1 = nch - 1
        g_wait(bsA, brA, gsA, grA, c0)
        add_rows(bsA, brA)
        o_start(bsA, oA, c0)
        g_wait(bsB, brB, gsB, grB, c1)
        add_rows(bsB, brB)
        o_start(bsB, oB, c1)
        o_wait(bsA, oA, c0)
        o_wait(bsB, oB, c1)

    return k(ps, pr, ix_s3, ix_r3)


def _sc_scatter(w0, w1, ex0, ex1, ix_r3):
    mesh = plsc.VectorSubcoreMesh(core_axis_name="c", subcore_axis_name="s")

    @functools.partial(
        pl.kernel,
        out_type=(jax.ShapeDtypeStruct((N_SP, 128), F32),
                  jax.ShapeDtypeStruct((N_SP, 128), F32),
                  jax.ShapeDtypeStruct((N_SP,), F32),
                  jax.ShapeDtypeStruct((N_SP,), F32)),
        mesh=mesh,
        scratch_types=[
            pltpu.VMEM_SHARED((N_SP, 128), F32),
            pltpu.VMEM_SHARED((N_SP,), F32),
            pltpu.VMEM((CH_PER_W, CHUNK), jnp.int32),
            pltpu.VMEM((E_PER_W,), F32),
            pltpu.VMEM((CHUNK, 128), F32),
            pltpu.VMEM((CHUNK, 128), F32),
            [pltpu.SemaphoreType.DMA] * 4,
        ],
    )
    def k(w0_hbm, w1_hbm, ex0_hbm, ex1_hbm, ix_hbm,
          a0_hbm, a1_hbm, s0_hbm, s1_hbm,
          spa, sps, ix_v, ex_v, bufA, bufB, sems):
        lA, lB, sA, sB = sems
        cid = lax.axis_index("c")
        sid = lax.axis_index("s")

        def zrow(i, carry):
            for j in range(8):
                bufA[i, pl.ds(j * 16, 16)] = jnp.zeros((16,), F32)
            return carry

        lax.fori_loop(0, CHUNK, zrow, 0)
        z0 = sid * ZROWS
        for kk in range(ZROWS // CHUNK):
            pltpu.sync_copy(bufA, spa.at[pl.ds(z0 + kk * CHUNK, CHUNK)])
            pltpu.sync_copy(bufA.at[0], sps.at[pl.ds(z0 + kk * CHUNK, CHUNK)])
        plsc.subcore_barrier()

        pltpu.sync_copy(ix_hbm.at[sid, pl.ds(cid * CH_PER_W, CH_PER_W)], ix_v)
        base = (sid * SID_CH + cid * CH_PER_W) * CHUNK

        def run(w_hbm, ex_hbm, lbase):
            pltpu.sync_copy(ex_hbm.at[pl.ds(lbase, E_PER_W)], ex_v)

            def l_start(buf, sem, c):
                pltpu.async_copy(
                    w_hbm.at[pl.ds(lbase + c * CHUNK, CHUNK)], buf, sem)

            def l_wait(buf, sem, c):
                pltpu.make_async_copy(
                    w_hbm.at[pl.ds(lbase + c * CHUNK, CHUNK)], buf, sem).wait()

            def s_start(buf, sem, c):
                pltpu.async_copy(buf, spa.at[ix_v.at[c]], sem, add=True)
                pltpu.async_copy(
                    ex_v.at[pl.ds(c * CHUNK, CHUNK)], sps.at[ix_v.at[c]],
                    sem, add=True)

            def s_wait(buf, sem, c):
                pltpu.make_async_copy(buf, spa.at[ix_v.at[c]], sem).wait()
                pltpu.make_async_copy(
                    ex_v.at[pl.ds(c * CHUNK, CHUNK)], sps.at[ix_v.at[c]],
                    sem).wait()

            l_start(bufA, lA, 0)
            l_start(bufB, lB, 1)

            def step(j, carry):
                c0 = 2 * j
                c1 = c0 + 1
                l_wait(bufA, lA, c0)
                s_start(bufA, sA, c0)
                l_wait(bufB, lB, c1)
                s_start(bufB, sB, c1)
                s_wait(bufA, sA, c0)
                l_start(bufA, lA, c0 + 2)
                s_wait(bufB, sB, c1)
                l_start(bufB, lB, c1 + 2)
                return carry

            lax.fori_loop(0, CH_PER_W // 2 - 1, step, 0)
            c0 = CH_PER_W - 2
            c1 = CH_PER_W - 1
            l_wait(bufA, lA, c0)
            s_start(bufA, sA, c0)
            l_wait(bufB, lB, c1)
            s_start(bufB, sB, c1)
            s_wait(bufA, sA, c0)
            s_wait(bufB, sB, c1)

        @pl.when(sid < 8)
        def _():
            run(w0_hbm, ex0_hbm, base)

        @pl.when(sid >= 8)
        def _():
            run(w1_hbm, ex1_hbm, base - E_H)

        plsc.subcore_barrier()

        @pl.when(cid == 0)
        def _():
            pltpu.sync_copy(spa.at[pl.ds(z0, ZROWS)], a0_hbm.at[pl.ds(z0, ZROWS)])
            pltpu.sync_copy(sps.at[pl.ds(z0, ZROWS)], s0_hbm.at[pl.ds(z0, ZROWS)])

        @pl.when(cid == 1)
        def _():
            pltpu.sync_copy(spa.at[pl.ds(z0, ZROWS)], a1_hbm.at[pl.ds(z0, ZROWS)])
            pltpu.sync_copy(sps.at[pl.ds(z0, ZROWS)], s1_hbm.at[pl.ds(z0, ZROWS)])

    return k(w0, w1, ex0, ex1, ix_r3)


def kernel(node_features, edge_features, senders, receivers, image_feature,
           params):
    pad = E_PAD - N_EDGES
    s_pad = jnp.concatenate([senders, jnp.zeros((pad,), jnp.int32)])
    r_pad = jnp.concatenate([receivers, jnp.full((pad,), N_NODES, jnp.int32)])
    ix_s4 = s_pad.reshape(2, 16, SID_CH_H, CHUNK)
    ix_r4 = r_pad.reshape(2, 16, SID_CH_H, CHUNK)
    ix_r3 = r_pad.reshape(16, SID_CH, CHUNK)

    node_lat = _enc_node(node_features, image_feature, params["enc_node"])
    edge_lat = _enc_edge(edge_features, params["enc_edge"])

    for i, blk in enumerate(params["blocks"]):
        ps, pr = _proj(node_lat, blk)
        pre0 = _sc_gather(ps, pr, ix_s4[0], ix_r4[0])
        pre1 = _sc_gather(ps, pr, ix_s4[1], ix_r4[1])
        e0 = elat0 if i else edge_lat
        e1 = elat1 if i else edge_lat
        outs0 = _edge_tc(pre0, e0, blk, want_resid=(i == 0),
                         e_off=0)
        outs1 = _edge_tc(pre1, e1, blk, want_resid=(i == 0),
                         e_off=0 if i else E_H // EB)
        if i == 0:
            elat0, elat1 = outs0[2], outs1[2]
        a0, a1, ss0, ss1 = _sc_scatter(
            outs0[0], outs1[0],
            outs0[1].reshape(E_H), outs1[1].reshape(E_H), ix_r3)
        node_lat = _node_tc(node_lat, a0, a1,
                            ss0.reshape(N_SP, 1), ss1.reshape(N_SP, 1),
                            blk["node"])

    dec = _dec_tc(node_lat, params["dec"])
    return dec[:, :3].reshape(1, N_NODES, 3)


# speedup vs baseline: 4.6593x; 1.0011x over previous
import functools

import jax
import jax.numpy as jnp
from jax import lax
from jax.experimental import pallas as pl
from jax.experimental.pallas import tpu as pltpu
from jax.experimental.pallas import tpu_sc as plsc

N_NODES = 10000
N_SP = 10240
N_EDGES = 160000
CHUNK = 128
CH_PER_W = 40
E_PER_W = CHUNK * CH_PER_W
E_PAD = 32 * E_PER_W
EB = 1280
NB = 2000
ZROWS = N_SP // 16
SID_CH = 80
E_H = E_PAD // 2
SID_CH_H = 40
GK0 = 30
F32 = jnp.float32
BF16 = jnp.bfloat16


def _ln_rows(x, g, beta):
    mu = jnp.mean(x, axis=-1, keepdims=True)
    var = jnp.mean((x - mu) ** 2, axis=-1, keepdims=True)
    return (x - mu) / jnp.sqrt(var + 1e-5) * g + beta


def _full(shape):
    nd = len(shape)
    return pl.BlockSpec(shape, lambda i: (0,) * nd)


def _enc_node(nf, img, p):
    (W1, b1), (W2, b2) = p["layers"]
    g, beta = p["ln"]
    W1a, W1b = W1[:128], W1[128:]

    def body(nf_ref, img_ref, w1a, w1b, b1_, w2, b2_, g_, be_, o_ref):
        h = nf_ref[...] @ w1a[...] + img_ref[...] @ w1b[...] + b1_[...]
        h = jnp.maximum(h, 0.0)
        y = h @ w2[...] + b2_[...]
        o_ref[...] = _ln_rows(y, g_[...], be_[...])

    return pl.pallas_call(
        body,
        grid=(N_NODES // NB,),
        in_specs=[
            pl.BlockSpec((NB, 128), lambda i: (i, 0)),
            _full((1, 512)), _full((128, 256)), _full((512, 256)),
            _full((256,)), _full((256, 128)), _full((128,)),
            _full((128,)), _full((128,)),
        ],
        out_specs=pl.BlockSpec((NB, 128), lambda i: (i, 0)),
        out_shape=jax.ShapeDtypeStruct((N_NODES, 128), F32),
    )(nf, img, W1a, W1b, b1, W2, b2, g, beta)


def _enc_edge(ef, p):
    (W1, b1), (W2, b2) = p["layers"]
    g, beta = p["ln"]

    def body(ef_ref, w1, b1_, w2, b2_, g_, be_, o_ref):
        h = ef_ref[...] @ w1[...] + b1_[...]
        h = jnp.maximum(h, 0.0)
        y = h @ w2[...] + b2_[...]
        o_ref[...] = _ln_rows(y, g_[...], be_[...])

    last = N_EDGES // EB - 1
    return pl.pallas_call(
        body,
        grid=(E_PAD // EB,),
        in_specs=[
            pl.BlockSpec((EB, 16), lambda i: (jnp.minimum(i, last), 0)),
            _full((16, 256)), _full((256,)), _full((256, 128)),
            _full((128,)), _full((128,)), _full((128,)),
        ],
        out_specs=pl.BlockSpec((EB, 128), lambda i: (i, 0)),
        out_shape=jax.ShapeDtypeStruct((E_PAD, 128), F32),
    )(ef, W1, b1, W2, b2, g, beta)


def _proj(nlat, blk):
    (W1, b1), _, _ = blk["edge"]["layers"]
    W1s, W1r = W1[:128], W1[128:256]

    def body(n_ref, w1s, w1r, b1_, ps_ref, pr_ref):
        n = n_ref[...]
        ps_ref[...] = n @ w1s[...] + b1_[...]
        pr_ref[...] = n @ w1r[...]

    return pl.pallas_call(
        body,
        grid=(N_NODES // NB,),
        in_specs=[
            pl.BlockSpec((NB, 128), lambda i: (i, 0)),
            _full((128, 128)), _full((128, 128)), _full((128,)),
        ],
        out_specs=[pl.BlockSpec((NB, 128), lambda i: (i, 0)),
                   pl.BlockSpec((NB, 128), lambda i: (i, 0))],
        out_shape=[jax.ShapeDtypeStruct((N_NODES, 128), F32),
                   jax.ShapeDtypeStruct((N_NODES, 128), F32)],
    )(nlat, W1s, W1r, b1)


def _edge_tc(pre, elat, blk, want_resid, e_off):
    _, (W2, b2), (W3, b3) = blk["edge"]["layers"]
    (W1, _), _, _ = blk["edge"]["layers"]
    g, beta = blk["edge"]["ln"]
    aW, ab = blk["att"]
    W1e = W1[256:].astype(BF16)
    aWr = aW.reshape(1, 128)
    ab2 = ab.reshape(1, 1)

    def body(p_ref, e_ref, w1e, b1_, w2, b2_, w3, b3_,
             g_, be_, aw, ab_, w_ref, x_ref, *res):
        e = e_ref[...]
        h = p_ref[...] + jnp.dot(e.astype(BF16), w1e[...],
                                 preferred_element_type=F32)
        h = jnp.maximum(h, 0.0)
        h = jnp.maximum(h @ w2[...] + b2_[...], 0.0)
        ne = _ln_rows(h @ w3[...] + b3_[...], g_[...], be_[...])
        lg = jnp.sum(ne * aw[...], axis=-1, keepdims=True) + ab_[...]
        lg = jnp.where(lg >= 0, lg, 0.2 * lg)
        ex = jnp.exp(lg)
        w_ref[...] = ne * ex
        x_ref[...] = ex
        if want_resid:
            res[0][...] = ne + e

    out_shape = [jax.ShapeDtypeStruct((E_H, 128), F32),
                 jax.ShapeDtypeStruct((E_H, 1), F32)]
    out_specs = [pl.BlockSpec((EB, 128), lambda i: (i, 0)),
                 pl.BlockSpec((EB, 1), lambda i: (i, 0))]
    if want_resid:
        out_shape.append(jax.ShapeDtypeStruct((E_H, 128), F32))
        out_specs.append(pl.BlockSpec((EB, 128), lambda i: (i, 0)))

    return pl.pallas_call(
        body,
        grid=(E_H // EB,),
        in_specs=[
            pl.BlockSpec((EB, 128), lambda i: (i, 0)),
            pl.BlockSpec((EB, 128), lambda i: (i + e_off, 0)),
            _full((128, 128)), _full((128,)),
            _full((128, 128)), _full((128,)),
            _full((128, 128)), _full((128,)),
            _full((128,)), _full((128,)), _full((1, 128)), _full((1, 1)),
        ],
        out_specs=out_specs,
        out_shape=out_shape,
    )(pre, elat, W1e, blk["edge"]["layers"][0][1], W2, b2, W3, b3, g, beta,
      aWr, ab2)


def _node_tc(nlat, a0, a1, s0, s1, p):
    (W1, b1), (W2, b2), (W3, b3) = p["layers"]
    g, beta = p["ln"]
    W1n, W1a = W1[:128], W1[128:]

    def body(n_ref, a0_, a1_, s0_, s1_, w1n, w1a, b1_, w2, b2_, w3, b3_,
             g_, be_, o_ref):
        n = n_ref[...]
        agg = (a0_[...] + a1_[...]) / (s0_[...] + s1_[...] + 1e-16)
        h = jnp.maximum(n @ w1n[...] + agg @ w1a[...] + b1_[...], 0.0)
        h = jnp.maximum(h @ w2[...] + b2_[...], 0.0)
        y = _ln_rows(h @ w3[...] + b3_[...], g_[...], be_[...])
        o_ref[...] = y + n

    return pl.pallas_call(
        body,
        grid=(N_NODES // NB,),
        in_specs=[
            pl.BlockSpec((NB, 128), lambda i: (i, 0)),
            pl.BlockSpec((NB, 128), lambda i: (i, 0)),
            pl.BlockSpec((NB, 128), lambda i: (i, 0)),
            pl.BlockSpec((NB, 1), lambda i: (i, 0)),
            pl.BlockSpec((NB, 1), lambda i: (i, 0)),
            _full((128, 128)), _full((128, 128)), _full((128,)),
            _full((128, 128)), _full((128,)), _full((128, 128)),
            _full((128,)), _full((128,)), _full((128,)),
        ],
        out_specs=pl.BlockSpec((NB, 128), lambda i: (i, 0)),
        out_shape=jax.ShapeDtypeStruct((N_NODES, 128), F32),
    )(nlat, a0, a1, s0, s1, W1n, W1a, b1, W2, b2, W3, b3, g, beta)


def _dec_tc(nlat, p):
    (W1, b1), (W2, b2), (W3, b3) = p["layers"]
    W3p = jnp.zeros((128, 128), F32).at[:, :W3.shape[1]].set(W3)
    b3p = jnp.zeros((128,), F32).at[:b3.shape[0]].set(b3)

    def body(n_ref, w1, b1_, w2, b2_, w3, b3_, o_ref):
        h = jnp.maximum(n_ref[...] @ w1[...] + b1_[...], 0.0)
        h = jnp.maximum(h @ w2[...] + b2_[...], 0.0)
        o_ref[...] = h @ w3[...] + b3_[...]

    return pl.pallas_call(
        body,
        grid=(N_NODES // NB,),
        in_specs=[
            pl.BlockSpec((NB, 128), lambda i: (i, 0)),
            _full((128, 128)), _full((128,)), _full((128, 128)),
            _full((128,)), _full((128, 128)), _full((128,)),
        ],
        out_specs=pl.BlockSpec((NB, 128), lambda i: (i, 0)),
        out_shape=jax.ShapeDtypeStruct((N_NODES, 128), F32),
    )(nlat, W1, b1, W2, b2, W3p, b3p)


def _sc_gather(ps, pr, ix_s3, ix_r3):
    mesh = plsc.VectorSubcoreMesh(core_axis_name="c", subcore_axis_name="s")

    @functools.partial(
        pl.kernel,
        out_type=jax.ShapeDtypeStruct((E_H, 128), F32),
        mesh=mesh,
        compiler_params=pltpu.CompilerParams(use_tc_tiling_on_sc=True),
        scratch_types=[
            pltpu.VMEM((SID_CH_H, CHUNK), jnp.int32),
            pltpu.VMEM((SID_CH_H, CHUNK), jnp.int32),
            pltpu.VMEM((CHUNK, 128), F32),
            pltpu.VMEM((CHUNK, 128), F32),
            pltpu.VMEM((CHUNK, 128), F32),
            pltpu.VMEM((CHUNK, 128), F32),
            [pltpu.SemaphoreType.DMA] * 6,
        ],
    )
    def k(ps_hbm, pr_hbm, ixs_hbm, ixr_hbm, o_hbm,
          ixs_v, ixr_v, bsA, brA, bsB, brB, sems):
        gsA, grA, gsB, grB, oA, oB = sems
        cid = lax.axis_index("c")
        sid = lax.axis_index("s")
        pltpu.sync_copy(ixs_hbm.at[sid], ixs_v)
        pltpu.sync_copy(ixr_hbm.at[sid], ixr_v)
        loc0 = jnp.where(cid == 0, 0, GK0)
        nch = jnp.where(cid == 0, GK0, SID_CH_H - GK0)
        base = (sid * SID_CH_H + loc0) * CHUNK

        def g_start(bs, br, sem_s, sem_r, c):
            pltpu.async_copy(ps_hbm.at[ixs_v.at[loc0 + c]], bs, sem_s)
            pltpu.async_copy(pr_hbm.at[ixr_v.at[loc0 + c]], br, sem_r)

        def g_wait(bs, br, sem_s, sem_r, c):
            pltpu.make_async_copy(
                ps_hbm.at[ixs_v.at[loc0 + c]], bs, sem_s).wait()
            pltpu.make_async_copy(
                pr_hbm.at[ixr_v.at[loc0 + c]], br, sem_r).wait()

        def add_rows(bs, br):
            def row(i, carry):
                for kk in range(8):
                    sl = pl.ds(kk * 16, 16)
                    bs[i, sl] = bs[i, sl] + br[i, sl]
                return carry
            lax.fori_loop(0, CHUNK, row, 0)

        def o_start(bs, sem, c):
            pltpu.async_copy(
                bs, o_hbm.at[pl.ds(base + c * CHUNK, CHUNK)], sem)

        def o_wait(bs, sem, c):
            pltpu.make_async_copy(
                bs, o_hbm.at[pl.ds(base + c * CHUNK, CHUNK)], sem).wait()

        g_start(bsA, brA, gsA, grA, 0)
        g_start(bsB, brB, gsB, grB, 1)

        def body(j, carry):
            c0 = 2 * j
            c1 = c0 + 1
            g_wait(bsA, brA, gsA, grA, c0)
            add_rows(bsA, brA)
            o_start(bsA, oA, c0)
            g_wait(bsB, brB, gsB, grB, c1)
            add_rows(bsB, brB)
            o_start(bsB, oB, c1)
            o_wait(bsA, oA, c0)
            g_start(bsA, brA, gsA, grA, c0 + 2)
            o_wait(bsB, oB, c1)
            g_start(bsB, brB, gsB, grB, c1 + 2)
            return carry

        lax.fori_loop(0, nch // 2 - 1, body, 0)
        c0 = nch - 2
        c1 = nch - 1
        g_wait(bsA, brA, gsA, grA, c0)
        add_rows(bsA, brA)
        o_start(bsA, oA, c0)
        g_wait(bsB, brB, gsB, grB, c1)
        add_rows(bsB, brB)
        o_start(bsB, oB, c1)
        o_wait(bsA, oA, c0)
        o_wait(bsB, oB, c1)

    return k(ps, pr, ix_s3, ix_r3)


def _sc_scatter(w0, w1, ex0, ex1, ix_r3):
    mesh = plsc.VectorSubcoreMesh(core_axis_name="c", subcore_axis_name="s")

    @functools.partial(
        pl.kernel,
        out_type=(jax.ShapeDtypeStruct((N_SP, 128), F32),
                  jax.ShapeDtypeStruct((N_SP, 128), F32),
                  jax.ShapeDtypeStruct((N_SP,), F32),
                  jax.ShapeDtypeStruct((N_SP,), F32)),
        mesh=mesh,
        compiler_params=pltpu.CompilerParams(use_tc_tiling_on_sc=True),
        scratch_types=[
            pltpu.VMEM_SHARED((N_SP, 128), F32),
            pltpu.VMEM_SHARED((N_SP,), F32),
            pltpu.VMEM((CH_PER_W, CHUNK), jnp.int32),
            pltpu.VMEM((E_PER_W,), F32),
            pltpu.VMEM((CHUNK, 128), F32),
            pltpu.VMEM((CHUNK, 128), F32),
            [pltpu.SemaphoreType.DMA] * 4,
        ],
    )
    def k(w0_hbm, w1_hbm, ex0_hbm, ex1_hbm, ix_hbm,
          a0_hbm, a1_hbm, s0_hbm, s1_hbm,
          spa, sps, ix_v, ex_v, bufA, bufB, sems):
        lA, lB, sA, sB = sems
        cid = lax.axis_index("c")
        sid = lax.axis_index("s")

        def zrow(i, carry):
            for j in range(8):
                bufA[i, pl.ds(j * 16, 16)] = jnp.zeros((16,), F32)
            return carry

        lax.fori_loop(0, CHUNK, zrow, 0)
        z0 = sid * ZROWS
        for kk in range(ZROWS // CHUNK):
            pltpu.sync_copy(bufA, spa.at[pl.ds(z0 + kk * CHUNK, CHUNK)])
            pltpu.sync_copy(bufA.at[0], sps.at[pl.ds(z0 + kk * CHUNK, CHUNK)])
        plsc.subcore_barrier()

        pltpu.sync_copy(ix_hbm.at[sid, pl.ds(cid * CH_PER_W, CH_PER_W)], ix_v)
        base = (sid * SID_CH + cid * CH_PER_W) * CHUNK

        def run(w_hbm, ex_hbm, lbase):
            pltpu.sync_copy(ex_hbm.at[pl.ds(lbase, E_PER_W)], ex_v)

            def l_start(buf, sem, c):
                pltpu.async_copy(
                    w_hbm.at[pl.ds(lbase + c * CHUNK, CHUNK)], buf, sem)

            def l_wait(buf, sem, c):
                pltpu.make_async_copy(
                    w_hbm.at[pl.ds(lbase + c * CHUNK, CHUNK)], buf, sem).wait()

            def s_start(buf, sem, c):
                pltpu.async_copy(buf, spa.at[ix_v.at[c]], sem, add=True)
                pltpu.async_copy(
                    ex_v.at[pl.ds(c * CHUNK, CHUNK)], sps.at[ix_v.at[c]],
                    sem, add=True)

            def s_wait(buf, sem, c):
                pltpu.make_async_copy(buf, spa.at[ix_v.at[c]], sem).wait()
                pltpu.make_async_copy(
                    ex_v.at[pl.ds(c * CHUNK, CHUNK)], sps.at[ix_v.at[c]],
                    sem).wait()

            l_start(bufA, lA, 0)
            l_start(bufB, lB, 1)

            def step(j, carry):
                c0 = 2 * j
                c1 = c0 + 1
                l_wait(bufA, lA, c0)
                s_start(bufA, sA, c0)
                l_wait(bufB, lB, c1)
                s_start(bufB, sB, c1)
                s_wait(bufA, sA, c0)
                l_start(bufA, lA, c0 + 2)
                s_wait(bufB, sB, c1)
                l_start(bufB, lB, c1 + 2)
                return carry

            lax.fori_loop(0, CH_PER_W // 2 - 1, step, 0)
            c0 = CH_PER_W - 2
            c1 = CH_PER_W - 1
            l_wait(bufA, lA, c0)
            s_start(bufA, sA, c0)
            l_wait(bufB, lB, c1)
            s_start(bufB, sB, c1)
            s_wait(bufA, sA, c0)
            s_wait(bufB, sB, c1)

        @pl.when(sid < 8)
        def _():
            run(w0_hbm, ex0_hbm, base)

        @pl.when(sid >= 8)
        def _():
            run(w1_hbm, ex1_hbm, base - E_H)

        plsc.subcore_barrier()

        @pl.when(cid == 0)
        def _():
            pltpu.sync_copy(spa.at[pl.ds(z0, ZROWS)], a0_hbm.at[pl.ds(z0, ZROWS)])
            pltpu.sync_copy(sps.at[pl.ds(z0, ZROWS)], s0_hbm.at[pl.ds(z0, ZROWS)])

        @pl.when(cid == 1)
        def _():
            pltpu.sync_copy(spa.at[pl.ds(z0, ZROWS)], a1_hbm.at[pl.ds(z0, ZROWS)])
            pltpu.sync_copy(sps.at[pl.ds(z0, ZROWS)], s1_hbm.at[pl.ds(z0, ZROWS)])

    return k(w0, w1, ex0, ex1, ix_r3)


def kernel(node_features, edge_features, senders, receivers, image_feature,
           params):
    pad = E_PAD - N_EDGES
    s_pad = jnp.concatenate([senders, jnp.zeros((pad,), jnp.int32)])
    r_pad = jnp.concatenate([receivers, jnp.full((pad,), N_NODES, jnp.int32)])
    ix_s4 = s_pad.reshape(2, 16, SID_CH_H, CHUNK)
    ix_r4 = r_pad.reshape(2, 16, SID_CH_H, CHUNK)
    ix_r3 = r_pad.reshape(16, SID_CH, CHUNK)

    node_lat = _enc_node(node_features, image_feature, params["enc_node"])
    edge_lat = _enc_edge(edge_features, params["enc_edge"])

    for i, blk in enumerate(params["blocks"]):
        ps, pr = _proj(node_lat, blk)
        pre0 = _sc_gather(ps, pr, ix_s4[0], ix_r4[0])
        pre1 = _sc_gather(ps, pr, ix_s4[1], ix_r4[1])
        e0 = elat0 if i else edge_lat
        e1 = elat1 if i else edge_lat
        outs0 = _edge_tc(pre0, e0, blk, want_resid=(i == 0),
                         e_off=0)
        outs1 = _edge_tc(pre1, e1, blk, want_resid=(i == 0),
                         e_off=0 if i else E_H // EB)
        if i == 0:
            elat0, elat1 = outs0[2], outs1[2]
        a0, a1, ss0, ss1 = _sc_scatter(
            outs0[0], outs1[0],
            outs0[1].reshape(E_H), outs1[1].reshape(E_H), ix_r3)
        node_lat = _node_tc(node_lat, a0, a1,
                            ss0.reshape(N_SP, 1), ss1.reshape(N_SP, 1),
                            blk["node"])

    dec = _dec_tc(node_lat, params["dec"])
    return dec[:, :3].reshape(1, N_NODES, 3)
